# baseline (speedup 1.0000x reference)
import functools

import jax
import jax.numpy as jnp
from jax import lax
from jax.experimental import pallas as pl

H = 128
PREC = lax.Precision.HIGHEST


def _fused_mlp(groups, adds, b1, W2, b2, lng, lnb, res, projs, block_rows):
    n_groups = len(groups)
    xs_counts = [len(xs) for xs, _ in groups]
    n_adds = len(adds)
    has_res = res is not None
    ln = lng is not None
    n_projs = len(projs)

    some_x = groups[0][0][0] if groups else adds[0]
    N = some_x.shape[0]
    dout = W2.shape[1]

    def body(*refs):
        it = iter(refs)
        u = None
        for gi in range(n_groups):
            xs = [next(it)[...] for _ in range(xs_counts[gi])]
            W = next(it)[...]
            x = xs[0]
            for extra in xs[1:]:
                x = x + extra
            t = jnp.dot(x, W, preferred_element_type=jnp.float32,
                        precision=PREC)
            u = t if u is None else u + t
        for _ in range(n_adds):
            a = next(it)[...]
            u = a if u is None else u + a
        b1v = next(it)[...]
        W2v = next(it)[...]
        b2v = next(it)[...]
        u = u + b1v
        h = u * jax.nn.sigmoid(u)
        z = jnp.dot(h, W2v, preferred_element_type=jnp.float32,
                    precision=PREC) + b2v
        if ln:
            gv = next(it)[...]
            bv = next(it)[...]
            mu = jnp.mean(z, axis=-1, keepdims=True)
            zc = z - mu
            var = jnp.mean(zc * zc, axis=-1, keepdims=True)
            z = zc * lax.rsqrt(var + 1e-5) * gv + bv
        if has_res:
            z = next(it)[...] + z
        pws = [next(it)[...] for _ in range(n_projs)]
        outs = list(it)
        outs[0][...] = z
        for k in range(n_projs):
            outs[1 + k][...] = jnp.dot(z, pws[k],
                                       preferred_element_type=jnp.float32,
                                       precision=PREC)

    inputs = []
    in_specs = []

    def add_rowblocked(a):
        inputs.append(a)
        in_specs.append(pl.BlockSpec((block_rows, a.shape[1]),
                                     lambda i: (i, 0)))

    def add_full(a):
        inputs.append(a)
        in_specs.append(pl.BlockSpec(a.shape, lambda i: (0,) * a.ndim))

    for xs, W in groups:
        for x in xs:
            add_rowblocked(x)
        add_full(W)
    for a in adds:
        add_rowblocked(a)
    add_full(b1.reshape(1, -1))
    add_full(W2)
    add_full(b2.reshape(1, -1))
    if ln:
        add_full(lng.reshape(1, -1))
        add_full(lnb.reshape(1, -1))
    if has_res:
        add_rowblocked(res)
    for Pw in projs:
        add_full(Pw)

    out_shapes = [jax.ShapeDtypeStruct((N, dout), jnp.float32)]
    out_shapes += [jax.ShapeDtypeStruct((N, H), jnp.float32)
                   for _ in range(n_projs)]
    out_specs = [pl.BlockSpec((block_rows, dout), lambda i: (i, 0))]
    out_specs += [pl.BlockSpec((block_rows, H), lambda i: (i, 0))
                  for _ in range(n_projs)]

    outs = pl.pallas_call(
        body,
        grid=(pl.cdiv(N, block_rows),),
        in_specs=in_specs,
        out_specs=out_specs,
        out_shape=out_shapes,
    )(*inputs)
    return outs if n_projs else outs[0]


def _mlp(p, x, ln=True, res=None, adds=(), extra_groups=(), projs=(),
         block_rows=1024):
    groups = [([x], p["W1"])] + list(extra_groups)
    lng = p["g"] if ln else None
    lnb = p["b"] if ln else None
    return _fused_mlp(groups, list(adds), p["b1"], p["W2"], p["b2"],
                      lng, lnb, res, list(projs), block_rows)


def _sc_gather_combine(S, D, src, dst):
    return S[src] + D[dst]


def _sc_scatter_add(e, dst, n_nodes):
    agg = jax.ops.segment_sum(e, dst, num_segments=n_nodes + 1)
    return [agg[:n_nodes]]


def _pad_edges(efeat, src, dst, n_dst, e_pad):
    e = efeat.shape[0]
    pad = e_pad - e
    efeat = jnp.pad(efeat, ((0, pad), (0, 0)))
    src = jnp.pad(src, (0, pad))
    dst = jnp.pad(dst, (0, pad), constant_values=n_dst)
    return efeat, src, dst


def _w1_split(p):
    W1 = p["W1"]
    return W1[:H], W1[H:2 * H], W1[2 * H:]


L = 4
N_MESH_ = 10000
N_GRID_ = 50000


def kernel(grid_nfeat, mesh_nfeat, g2m_efeat, mesh_efeat, m2g_efeat,
           g2m_src, g2m_dst, mesh_src, mesh_dst, m2g_src, m2g_dst, params):
    P = params

    EP_G2M = 200704
    EP_MESH = 163840
    EP_M2G = 151552

    g2m_efeat, g2m_src, g2m_dst = _pad_edges(
        g2m_efeat, g2m_src, g2m_dst, N_MESH_, EP_G2M)
    mesh_efeat, mesh_src, mesh_dst = _pad_edges(
        mesh_efeat, mesh_src, mesh_dst, N_MESH_, EP_MESH)
    m2g_efeat, m2g_src, m2g_dst = _pad_edges(
        m2g_efeat, m2g_src, m2g_dst, N_GRID_, EP_M2G)

    W1e_g2m, W1s_g2m, W1d_g2m = _w1_split(P["g2m_edge_mlp"])
    W1e_m2g, W1s_m2g, W1d_m2g = _w1_split(P["m2g_edge_mlp"])
    proc_e = [_w1_split(P["proc_edge_%d" % i]) for i in range(L)]

    g, S_g2m = _mlp(P["grid_enc"], grid_nfeat, projs=(W1s_g2m,))
    m, D_g2m = _mlp(P["mesh_enc"], mesh_nfeat, projs=(W1d_g2m,))
    e_g2m = _mlp(P["g2m_edge_enc"], g2m_efeat)
    e_mesh = _mlp(P["mesh_edge_enc"], mesh_efeat)
    e_m2g = _mlp(P["m2g_edge_enc"], m2g_efeat)

    G = _sc_gather_combine(S_g2m, D_g2m, g2m_src, g2m_dst)
    gp_edge = dict(P["g2m_edge_mlp"], W1=W1e_g2m)
    e_g2m = _mlp(gp_edge, e_g2m, adds=(G,), res=e_g2m)
    aggs = _sc_scatter_add(e_g2m, g2m_dst, N_MESH_)
    nodep = P["g2m_node_mlp"]
    W1m, W1a = nodep["W1"][:H], nodep["W1"][H:]
    m, S0, D0 = _fused_mlp(
        [([m], W1m), (aggs, W1a)], [], nodep["b1"], nodep["W2"], nodep["b2"],
        nodep["g"], nodep["b"], m, [proc_e[0][1], proc_e[0][2]], 1024)

    g, D_m2g = _mlp(P["enc_grid_mlp"], g, res=g, projs=(W1d_m2g,))

    S, D = S0, D0
    for i in range(L):
        G = _sc_gather_combine(S, D, mesh_src, mesh_dst)
        ep = P["proc_edge_%d" % i]
        ep_edge = dict(ep, W1=proc_e[i][0])
        e_mesh = _mlp(ep_edge, e_mesh, adds=(G,), res=e_mesh)
        aggs = _sc_scatter_add(e_mesh, mesh_dst, N_MESH_)
        np_ = P["proc_node_%d" % i]
        W1m, W1a = np_["W1"][:H], np_["W1"][H:]
        if i + 1 < L:
            projs = [proc_e[i + 1][1], proc_e[i + 1][2]]
        else:
            projs = [W1s_m2g]
        outs = _fused_mlp(
            [([m], W1m), (aggs, W1a)], [], np_["b1"], np_["W2"], np_["b2"],
            np_["g"], np_["b"], m, projs, 1024)
        if i + 1 < L:
            m, S, D = outs
        else:
            m, S_m2g = outs

    G = _sc_gather_combine(S_m2g, D_m2g, m2g_src, m2g_dst)
    dp = P["m2g_edge_mlp"]
    dp_edge = dict(dp, W1=W1e_m2g)
    e_m2g = _mlp(dp_edge, e_m2g, adds=(G,), res=e_m2g)
    aggs = _sc_scatter_add(e_m2g, m2g_dst, N_GRID_)
    decp = P["dec_node_mlp"]
    W1g, W1a = decp["W1"][:H], decp["W1"][H:]
    g = _fused_mlp(
        [([g], W1g), (aggs, W1a)], [], decp["b1"], decp["W2"], decp["b2"],
        decp["g"], decp["b"], g, [], 1024)

    return _mlp(P["final_mlp"], g, ln=False)


# speedup vs baseline: 1.9970x; 1.9970x over previous
import functools

import jax
import jax.numpy as jnp
from jax import lax
from jax.experimental import pallas as pl
from jax.experimental.pallas import tpu as pltpu
from jax.experimental.pallas import tpu_sc as plsc

H = 128
PREC = lax.Precision.HIGHEST
NC = 2
NS = 16
NW = NC * NS
TB = 128


def _fused_mlp(groups, adds, b1, W2, b2, lng, lnb, res, projs, block_rows):
    n_groups = len(groups)
    xs_counts = [len(xs) for xs, _ in groups]
    n_adds = len(adds)
    has_res = res is not None
    ln = lng is not None
    n_projs = len(projs)

    some_x = groups[0][0][0] if groups else adds[0]
    N = some_x.shape[0]
    dout = W2.shape[1]

    def body(*refs):
        it = iter(refs)
        u = None
        for gi in range(n_groups):
            xs = [next(it)[...] for _ in range(xs_counts[gi])]
            W = next(it)[...]
            x = xs[0]
            for extra in xs[1:]:
                x = x + extra
            t = jnp.dot(x, W, preferred_element_type=jnp.float32,
                        precision=PREC)
            u = t if u is None else u + t
        for _ in range(n_adds):
            a = next(it)[...]
            u = a if u is None else u + a
        b1v = next(it)[...]
        W2v = next(it)[...]
        b2v = next(it)[...]
        u = u + b1v
        h = u * jax.nn.sigmoid(u)
        z = jnp.dot(h, W2v, preferred_element_type=jnp.float32,
                    precision=PREC) + b2v
        if ln:
            gv = next(it)[...]
            bv = next(it)[...]
            mu = jnp.mean(z, axis=-1, keepdims=True)
            zc = z - mu
            var = jnp.mean(zc * zc, axis=-1, keepdims=True)
            z = zc * lax.rsqrt(var + 1e-5) * gv + bv
        if has_res:
            z = next(it)[...] + z
        pws = [next(it)[...] for _ in range(n_projs)]
        outs = list(it)
        outs[0][...] = z
        for k in range(n_projs):
            outs[1 + k][...] = jnp.dot(z, pws[k],
                                       preferred_element_type=jnp.float32,
                                       precision=PREC)

    inputs = []
    in_specs = []

    def add_rowblocked(a):
        inputs.append(a)
        in_specs.append(pl.BlockSpec((block_rows, a.shape[1]),
                                     lambda i: (i, 0)))

    def add_full(a):
        inputs.append(a)
        in_specs.append(pl.BlockSpec(a.shape, lambda i: (0,) * a.ndim))

    for xs, W in groups:
        for x in xs:
            add_rowblocked(x)
        add_full(W)
    for a in adds:
        add_rowblocked(a)
    add_full(b1.reshape(1, -1))
    add_full(W2)
    add_full(b2.reshape(1, -1))
    if ln:
        add_full(lng.reshape(1, -1))
        add_full(lnb.reshape(1, -1))
    if has_res:
        add_rowblocked(res)
    for Pw in projs:
        add_full(Pw)

    out_shapes = [jax.ShapeDtypeStruct((N, dout), jnp.float32)]
    out_shapes += [jax.ShapeDtypeStruct((N, H), jnp.float32)
                   for _ in range(n_projs)]
    out_specs = [pl.BlockSpec((block_rows, dout), lambda i: (i, 0))]
    out_specs += [pl.BlockSpec((block_rows, H), lambda i: (i, 0))
                  for _ in range(n_projs)]

    outs = pl.pallas_call(
        body,
        grid=(pl.cdiv(N, block_rows),),
        in_specs=in_specs,
        out_specs=out_specs,
        out_shape=out_shapes,
    )(*inputs)
    return outs if n_projs else outs[0]


def _mlp(p, x, ln=True, res=None, adds=(), extra_groups=(), projs=(),
         block_rows=1024):
    groups = [([x], p["W1"])] + list(extra_groups)
    lng = p["g"] if ln else None
    lnb = p["b"] if ln else None
    return _fused_mlp(groups, list(adds), p["b1"], p["W2"], p["b2"],
                      lng, lnb, res, list(projs), block_rows)


def _sc_gather_combine(S, D, src, dst):
    E = src.shape[0]
    per_w = E // NW
    n_tiles = per_w // TB
    mesh = plsc.VectorSubcoreMesh(core_axis_name="c", subcore_axis_name="s")

    @functools.partial(
        pl.kernel, mesh=mesh,
        out_type=[jax.ShapeDtypeStruct((E, H), jnp.float32),
                  jax.ShapeDtypeStruct((E, H), jnp.float32)],
        scratch_types=[
            pltpu.VMEM((TB,), jnp.int32),
            pltpu.VMEM((TB,), jnp.int32),
            pltpu.VMEM((TB, H), jnp.float32),
            pltpu.VMEM((TB, H), jnp.float32),
            pltpu.SemaphoreType.DMA,
            pltpu.SemaphoreType.DMA,
        ],
    )
    def k(S_hbm, D_hbm, src_hbm, dst_hbm, gs_hbm, gd_hbm,
          sidx, didx, srows, drows, sem1, sem2):
        wid = lax.axis_index("s") * NC + lax.axis_index("c")
        base = wid * per_w

        def step(t, carry):
            off = base + t * TB
            pltpu.sync_copy(src_hbm.at[pl.ds(off, TB)], sidx)
            pltpu.sync_copy(dst_hbm.at[pl.ds(off, TB)], didx)
            c1 = pltpu.async_copy(S_hbm.at[sidx], srows, sem1)
            c2 = pltpu.async_copy(D_hbm.at[didx], drows, sem2)
            c1.wait()
            c2.wait()
            pltpu.sync_copy(srows, gs_hbm.at[pl.ds(off, TB)])
            pltpu.sync_copy(drows, gd_hbm.at[pl.ds(off, TB)])
            return carry

        lax.fori_loop(0, n_tiles, step, 0)

    return k(S, D, src, dst)


def _sc_scatter_add(e, dst, n_nodes, n_chunks):
    E = e.shape[0]
    per_c = E // NC
    per_w = E // NW
    n_tiles = per_w // TB
    ch = -(-max(n_nodes + 1, 128) // (n_chunks * 128)) * 128
    stripe = ch // 16
    sp_rows = ch + 16
    mesh = plsc.VectorSubcoreMesh(core_axis_name="c", subcore_axis_name="s")
    zeros = jnp.zeros((ch, H), jnp.float32)

    @functools.partial(
        pl.kernel, mesh=mesh,
        out_type=jax.ShapeDtypeStruct((NC, n_chunks * ch, H), jnp.float32),
        scratch_types=[
            pltpu.VMEM((TB,), jnp.int32),
            pltpu.VMEM((TB,), jnp.int32),
            pltpu.VMEM((TB, H), jnp.float32),
            pltpu.VMEM_SHARED((sp_rows, H), jnp.float32),
        ],
    )
    def k(e_hbm, dst_hbm, z_hbm, out_hbm, didx, lidx, erows, acc):
        c = lax.axis_index("c")
        s = lax.axis_index("s")
        base = c * per_c + s * per_w
        for chunk in range(n_chunks):
            cbase = chunk * ch
            pltpu.sync_copy(z_hbm.at[pl.ds(s * stripe, stripe)],
                            acc.at[pl.ds(s * stripe, stripe)])
            plsc.subcore_barrier()

            def step(t, carry):
                off = base + t * TB
                pltpu.sync_copy(dst_hbm.at[pl.ds(off, TB)], didx)
                for j in range(TB // 16):
                    v = didx[pl.ds(j * 16, 16)]
                    inb = (v >= cbase) & (v < cbase + ch)
                    lidx[pl.ds(j * 16, 16)] = jnp.where(inb, v - cbase, ch)
                pltpu.sync_copy(e_hbm.at[pl.ds(off, TB)], erows)
                pltpu.sync_copy(erows, acc.at[lidx], add=True)
                return carry

            lax.fori_loop(0, n_tiles, step, 0)
            plsc.subcore_barrier()
            pltpu.sync_copy(
                acc.at[pl.ds(s * stripe, stripe)],
                out_hbm.at[c, pl.ds(cbase + s * stripe, stripe)])
            plsc.subcore_barrier()

    out = k(e, dst, zeros)
    return [out[0, :n_nodes], out[1, :n_nodes]]


def _pad_edges(efeat, src, dst, n_dst, e_pad):
    e = efeat.shape[0]
    pad = e_pad - e
    efeat = jnp.pad(efeat, ((0, pad), (0, 0)))
    src = jnp.pad(src, (0, pad))
    dst = jnp.pad(dst, (0, pad), constant_values=n_dst)
    return efeat, src, dst


def _w1_split(p):
    W1 = p["W1"]
    return W1[:H], W1[H:2 * H], W1[2 * H:]


L = 4
N_MESH_ = 10000
N_GRID_ = 50000


def kernel(grid_nfeat, mesh_nfeat, g2m_efeat, mesh_efeat, m2g_efeat,
           g2m_src, g2m_dst, mesh_src, mesh_dst, m2g_src, m2g_dst, params):
    P = params

    EP_G2M = 200704
    EP_MESH = 163840
    EP_M2G = 151552

    g2m_efeat, g2m_src, g2m_dst = _pad_edges(
        g2m_efeat, g2m_src, g2m_dst, N_MESH_, EP_G2M)
    mesh_efeat, mesh_src, mesh_dst = _pad_edges(
        mesh_efeat, mesh_src, mesh_dst, N_MESH_, EP_MESH)
    m2g_efeat, m2g_src, m2g_dst = _pad_edges(
        m2g_efeat, m2g_src, m2g_dst, N_GRID_, EP_M2G)

    W1e_g2m, W1s_g2m, W1d_g2m = _w1_split(P["g2m_edge_mlp"])
    W1e_m2g, W1s_m2g, W1d_m2g = _w1_split(P["m2g_edge_mlp"])
    proc_e = [_w1_split(P["proc_edge_%d" % i]) for i in range(L)]

    g, S_g2m = _mlp(P["grid_enc"], grid_nfeat, projs=(W1s_g2m,))
    m, D_g2m = _mlp(P["mesh_enc"], mesh_nfeat, projs=(W1d_g2m,))
    e_g2m = _mlp(P["g2m_edge_enc"], g2m_efeat)
    e_mesh = _mlp(P["mesh_edge_enc"], mesh_efeat)
    e_m2g = _mlp(P["m2g_edge_enc"], m2g_efeat)

    Gs, Gd = _sc_gather_combine(S_g2m, D_g2m, g2m_src, g2m_dst)
    gp_edge = dict(P["g2m_edge_mlp"], W1=W1e_g2m)
    e_g2m = _mlp(gp_edge, e_g2m, adds=(Gs, Gd), res=e_g2m)
    aggs = _sc_scatter_add(e_g2m, g2m_dst, N_MESH_, 1)
    nodep = P["g2m_node_mlp"]
    W1m, W1a = nodep["W1"][:H], nodep["W1"][H:]
    m, S0, D0 = _fused_mlp(
        [([m], W1m), (aggs, W1a)], [], nodep["b1"], nodep["W2"], nodep["b2"],
        nodep["g"], nodep["b"], m, [proc_e[0][1], proc_e[0][2]], 1024)

    g, D_m2g = _mlp(P["enc_grid_mlp"], g, res=g, projs=(W1d_m2g,))

    S, D = S0, D0
    for i in range(L):
        Gs, Gd = _sc_gather_combine(S, D, mesh_src, mesh_dst)
        ep = P["proc_edge_%d" % i]
        ep_edge = dict(ep, W1=proc_e[i][0])
        e_mesh = _mlp(ep_edge, e_mesh, adds=(Gs, Gd), res=e_mesh)
        aggs = _sc_scatter_add(e_mesh, mesh_dst, N_MESH_, 1)
        np_ = P["proc_node_%d" % i]
        W1m, W1a = np_["W1"][:H], np_["W1"][H:]
        if i + 1 < L:
            projs = [proc_e[i + 1][1], proc_e[i + 1][2]]
        else:
            projs = [W1s_m2g]
        outs = _fused_mlp(
            [([m], W1m), (aggs, W1a)], [], np_["b1"], np_["W2"], np_["b2"],
            np_["g"], np_["b"], m, projs, 1024)
        if i + 1 < L:
            m, S, D = outs
        else:
            m, S_m2g = outs

    Gs, Gd = _sc_gather_combine(S_m2g, D_m2g, m2g_src, m2g_dst)
    dp = P["m2g_edge_mlp"]
    dp_edge = dict(dp, W1=W1e_m2g)
    e_m2g = _mlp(dp_edge, e_m2g, adds=(Gs, Gd), res=e_m2g)
    aggs = _sc_scatter_add(e_m2g, m2g_dst, N_GRID_, 4)
    decp = P["dec_node_mlp"]
    W1g, W1a = decp["W1"][:H], decp["W1"][H:]
    g = _fused_mlp(
        [([g], W1g), (aggs, W1a)], [], decp["b1"], decp["W2"], decp["b2"],
        decp["g"], decp["b"], g, [], 1024)

    return _mlp(P["final_mlp"], g, ln=False)


# speedup vs baseline: 2.1244x; 1.0638x over previous
import functools

import jax
import jax.numpy as jnp
from jax import lax
from jax.experimental import pallas as pl
from jax.experimental.pallas import tpu as pltpu
from jax.experimental.pallas import tpu_sc as plsc

H = 128
PREC = lax.Precision.HIGHEST
NC = 2
NS = 16
NW = NC * NS
TB = 128


def _fused_mlp(groups, adds, b1, W2, b2, lng, lnb, res, projs, block_rows):
    n_groups = len(groups)
    xs_counts = [len(xs) for xs, _ in groups]
    n_adds = len(adds)
    has_res = res is not None
    ln = lng is not None
    n_projs = len(projs)

    some_x = groups[0][0][0] if groups else adds[0]
    N = some_x.shape[0]
    dout = W2.shape[1]

    def body(*refs):
        it = iter(refs)
        u = None
        for gi in range(n_groups):
            xs = [next(it)[...] for _ in range(xs_counts[gi])]
            W = next(it)[...]
            x = xs[0]
            for extra in xs[1:]:
                x = x + extra
            t = jnp.dot(x, W, preferred_element_type=jnp.float32,
                        precision=PREC)
            u = t if u is None else u + t
        for _ in range(n_adds):
            a = next(it)[...]
            u = a if u is None else u + a
        b1v = next(it)[...]
        W2v = next(it)[...]
        b2v = next(it)[...]
        u = u + b1v
        h = u * jax.nn.sigmoid(u)
        z = jnp.dot(h, W2v, preferred_element_type=jnp.float32,
                    precision=PREC) + b2v
        if ln:
            gv = next(it)[...]
            bv = next(it)[...]
            mu = jnp.mean(z, axis=-1, keepdims=True)
            zc = z - mu
            var = jnp.mean(zc * zc, axis=-1, keepdims=True)
            z = zc * lax.rsqrt(var + 1e-5) * gv + bv
        if has_res:
            z = next(it)[...] + z
        pws = [next(it)[...] for _ in range(n_projs)]
        outs = list(it)
        outs[0][...] = z
        for k in range(n_projs):
            outs[1 + k][...] = jnp.dot(z, pws[k],
                                       preferred_element_type=jnp.float32,
                                       precision=PREC)

    inputs = []
    in_specs = []

    def add_rowblocked(a):
        inputs.append(a)
        in_specs.append(pl.BlockSpec((block_rows, a.shape[1]),
                                     lambda i: (i, 0)))

    def add_full(a):
        inputs.append(a)
        in_specs.append(pl.BlockSpec(a.shape, lambda i: (0,) * a.ndim))

    for xs, W in groups:
        for x in xs:
            add_rowblocked(x)
        add_full(W)
    for a in adds:
        add_rowblocked(a)
    add_full(b1.reshape(1, -1))
    add_full(W2)
    add_full(b2.reshape(1, -1))
    if ln:
        add_full(lng.reshape(1, -1))
        add_full(lnb.reshape(1, -1))
    if has_res:
        add_rowblocked(res)
    for Pw in projs:
        add_full(Pw)

    out_shapes = [jax.ShapeDtypeStruct((N, dout), jnp.float32)]
    out_shapes += [jax.ShapeDtypeStruct((N, H), jnp.float32)
                   for _ in range(n_projs)]
    out_specs = [pl.BlockSpec((block_rows, dout), lambda i: (i, 0))]
    out_specs += [pl.BlockSpec((block_rows, H), lambda i: (i, 0))
                  for _ in range(n_projs)]

    outs = pl.pallas_call(
        body,
        grid=(pl.cdiv(N, block_rows),),
        in_specs=in_specs,
        out_specs=out_specs,
        out_shape=out_shapes,
    )(*inputs)
    return outs if n_projs else outs[0]


def _mlp(p, x, ln=True, res=None, adds=(), extra_groups=(), projs=(),
         block_rows=1024):
    groups = [([x], p["W1"])] + list(extra_groups)
    lng = p["g"] if ln else None
    lnb = p["b"] if ln else None
    return _fused_mlp(groups, list(adds), p["b1"], p["W2"], p["b2"],
                      lng, lnb, res, list(projs), block_rows)


def _sc_gather_combine(S, D, src, dst):
    E = src.shape[0]
    per_w = E // NW
    nt = per_w // TB
    mesh = plsc.VectorSubcoreMesh(core_axis_name="c", subcore_axis_name="s")

    @functools.partial(
        pl.kernel, mesh=mesh,
        out_type=[jax.ShapeDtypeStruct((E, H), jnp.float32),
                  jax.ShapeDtypeStruct((E, H), jnp.float32)],
        scratch_types=[
            pltpu.VMEM((per_w,), jnp.int32),
            pltpu.VMEM((per_w,), jnp.int32),
            pltpu.VMEM((2, TB, H), jnp.float32),
            pltpu.VMEM((2, TB, H), jnp.float32),
            pltpu.SemaphoreType.DMA,
            [pltpu.SemaphoreType.DMA] * 2,
            [pltpu.SemaphoreType.DMA] * 2,
        ],
    )
    def k(S_hbm, D_hbm, src_hbm, dst_hbm, gs_hbm, gd_hbm,
          sidx, didx, rs, rd, semi, semg, semo):
        wid = lax.axis_index("s") * NC + lax.axis_index("c")
        base = wid * per_w

        c1 = pltpu.async_copy(src_hbm.at[pl.ds(base, per_w)], sidx, semi)
        c2 = pltpu.async_copy(dst_hbm.at[pl.ds(base, per_w)], didx, semi)
        c1.wait()
        c2.wait()

        def issue_gather(t, slot):
            pltpu.async_copy(S_hbm.at[sidx.at[pl.ds(t * TB, TB)]],
                             rs.at[slot], semg[slot])
            pltpu.async_copy(D_hbm.at[didx.at[pl.ds(t * TB, TB)]],
                             rd.at[slot], semg[slot])

        def wait_gather(slot):
            pltpu.make_async_copy(S_hbm.at[pl.ds(0, TB)], rs.at[slot],
                                  semg[slot]).wait()
            pltpu.make_async_copy(D_hbm.at[pl.ds(0, TB)], rd.at[slot],
                                  semg[slot]).wait()

        def issue_store(t, slot):
            off = base + t * TB
            pltpu.async_copy(rs.at[slot], gs_hbm.at[pl.ds(off, TB)],
                             semo[slot])
            pltpu.async_copy(rd.at[slot], gd_hbm.at[pl.ds(off, TB)],
                             semo[slot])

        def wait_store(slot):
            pltpu.make_async_copy(rs.at[slot], gs_hbm.at[pl.ds(0, TB)],
                                  semo[slot]).wait()
            pltpu.make_async_copy(rd.at[slot], gd_hbm.at[pl.ds(0, TB)],
                                  semo[slot]).wait()

        issue_gather(0, 0)

        def pair(i, carry):
            t0 = 2 * i

            @pl.when(i > 0)
            def _():
                wait_store(1)

            issue_gather(t0 + 1, 1)
            wait_gather(0)
            issue_store(t0, 0)

            @pl.when(t0 + 2 < nt)
            def _():
                wait_store(0)
                issue_gather(t0 + 2, 0)

            wait_gather(1)
            issue_store(t0 + 1, 1)
            return carry

        lax.fori_loop(0, nt // 2, pair, 0)
        if nt % 2:
            wait_gather(0)
            issue_store(nt - 1, 0)
            wait_store(0)
            if nt > 1:
                wait_store(1)
        else:
            wait_store(0)
            wait_store(1)

    return k(S, D, src, dst)


def _sc_scatter_add(e, dst, n_nodes, n_chunks):
    E = e.shape[0]
    per_w = E // NW
    nt = per_w // TB
    ch = -(-max(n_nodes + 1, 128) // (n_chunks * 128)) * 128
    stripe = ch // 16
    sp_rows = ch + 16
    mesh = plsc.VectorSubcoreMesh(core_axis_name="c", subcore_axis_name="s")
    zeros = jnp.zeros((ch, H), jnp.float32)

    @functools.partial(
        pl.kernel, mesh=mesh,
        out_type=jax.ShapeDtypeStruct((NC, n_chunks * ch, H), jnp.float32),
        scratch_types=[
            pltpu.VMEM((per_w,), jnp.int32),
            pltpu.VMEM((nt, TB), jnp.int32),
            pltpu.VMEM((2, TB, H), jnp.float32),
            pltpu.VMEM_SHARED((sp_rows, H), jnp.float32),
            pltpu.SemaphoreType.DMA,
            [pltpu.SemaphoreType.DMA] * 2,
            [pltpu.SemaphoreType.DMA] * 2,
        ],
    )
    def k(e_hbm, dst_hbm, z_hbm, out_hbm, didx, lidx, er, acc,
          semi, seme, sema):
        c = lax.axis_index("c")
        s = lax.axis_index("s")
        wid = s * NC + c
        base = wid * per_w

        pltpu.async_copy(dst_hbm.at[pl.ds(base, per_w)], didx, semi).wait()

        def issue_load(t, slot):
            pltpu.async_copy(e_hbm.at[pl.ds(base + t * TB, TB)],
                             er.at[slot], seme[slot])

        def wait_load(slot):
            pltpu.make_async_copy(e_hbm.at[pl.ds(0, TB)], er.at[slot],
                                  seme[slot]).wait()

        def issue_scat(t, slot):
            pltpu.async_copy(er.at[slot], acc.at[lidx.at[t]], sema[slot],
                             add=True)

        def wait_scat(slot):
            pltpu.make_async_copy(er.at[slot], acc.at[pl.ds(0, TB)],
                                  sema[slot]).wait()

        for chunk in range(n_chunks):
            cbase = chunk * ch
            pltpu.sync_copy(z_hbm.at[pl.ds(s * stripe, stripe)],
                            acc.at[pl.ds(s * stripe, stripe)])

            def trans(t, carry):
                for j in range(TB // 16):
                    v = didx[pl.ds(t * TB + j * 16, 16)]
                    inb = (v >= cbase) & (v < cbase + ch)
                    lidx[t, pl.ds(j * 16, 16)] = jnp.where(inb, v - cbase, ch)
                return carry

            lax.fori_loop(0, nt, trans, 0)
            plsc.subcore_barrier()

            issue_load(0, 0)

            def pair(i, carry):
                t0 = 2 * i

                @pl.when(i > 0)
                def _():
                    wait_scat(1)

                issue_load(t0 + 1, 1)
                wait_load(0)
                issue_scat(t0, 0)

                @pl.when(t0 + 2 < nt)
                def _():
                    wait_scat(0)
                    issue_load(t0 + 2, 0)

                wait_load(1)
                issue_scat(t0 + 1, 1)
                return carry

            lax.fori_loop(0, nt // 2, pair, 0)
            if nt % 2:
                wait_load(0)
                issue_scat(nt - 1, 0)
                wait_scat(0)
                if nt > 1:
                    wait_scat(1)
            else:
                wait_scat(0)
                wait_scat(1)
            plsc.subcore_barrier()
            pltpu.sync_copy(
                acc.at[pl.ds(s * stripe, stripe)],
                out_hbm.at[c, pl.ds(cbase + s * stripe, stripe)])
            plsc.subcore_barrier()

    out = k(e, dst, zeros)
    return [out[0, :n_nodes], out[1, :n_nodes]]


def _pad_edges(efeat, src, dst, n_dst, e_pad):
    e = efeat.shape[0]
    pad = e_pad - e
    efeat = jnp.pad(efeat, ((0, pad), (0, 0)))
    src = jnp.pad(src, (0, pad))
    dst = jnp.pad(dst, (0, pad), constant_values=n_dst)
    return efeat, src, dst


def _w1_split(p):
    W1 = p["W1"]
    return W1[:H], W1[H:2 * H], W1[2 * H:]


L = 4
N_MESH_ = 10000
N_GRID_ = 50000


def kernel(grid_nfeat, mesh_nfeat, g2m_efeat, mesh_efeat, m2g_efeat,
           g2m_src, g2m_dst, mesh_src, mesh_dst, m2g_src, m2g_dst, params):
    P = params

    EP_G2M = 200704
    EP_MESH = 163840
    EP_M2G = 151552

    g2m_efeat, g2m_src, g2m_dst = _pad_edges(
        g2m_efeat, g2m_src, g2m_dst, N_MESH_, EP_G2M)
    mesh_efeat, mesh_src, mesh_dst = _pad_edges(
        mesh_efeat, mesh_src, mesh_dst, N_MESH_, EP_MESH)
    m2g_efeat, m2g_src, m2g_dst = _pad_edges(
        m2g_efeat, m2g_src, m2g_dst, N_GRID_, EP_M2G)

    W1e_g2m, W1s_g2m, W1d_g2m = _w1_split(P["g2m_edge_mlp"])
    W1e_m2g, W1s_m2g, W1d_m2g = _w1_split(P["m2g_edge_mlp"])
    proc_e = [_w1_split(P["proc_edge_%d" % i]) for i in range(L)]

    g, S_g2m = _mlp(P["grid_enc"], grid_nfeat, projs=(W1s_g2m,))
    m, D_g2m = _mlp(P["mesh_enc"], mesh_nfeat, projs=(W1d_g2m,))
    e_g2m = _mlp(P["g2m_edge_enc"], g2m_efeat)
    e_mesh = _mlp(P["mesh_edge_enc"], mesh_efeat)
    e_m2g = _mlp(P["m2g_edge_enc"], m2g_efeat)

    Gs, Gd = _sc_gather_combine(S_g2m, D_g2m, g2m_src, g2m_dst)
    gp_edge = dict(P["g2m_edge_mlp"], W1=W1e_g2m)
    e_g2m = _mlp(gp_edge, e_g2m, adds=(Gs, Gd), res=e_g2m)
    aggs = _sc_scatter_add(e_g2m, g2m_dst, N_MESH_, 1)
    nodep = P["g2m_node_mlp"]
    W1m, W1a = nodep["W1"][:H], nodep["W1"][H:]
    m, S0, D0 = _fused_mlp(
        [([m], W1m), (aggs, W1a)], [], nodep["b1"], nodep["W2"], nodep["b2"],
        nodep["g"], nodep["b"], m, [proc_e[0][1], proc_e[0][2]], 1024)

    g, D_m2g = _mlp(P["enc_grid_mlp"], g, res=g, projs=(W1d_m2g,))

    S, D = S0, D0
    for i in range(L):
        Gs, Gd = _sc_gather_combine(S, D, mesh_src, mesh_dst)
        ep = P["proc_edge_%d" % i]
        ep_edge = dict(ep, W1=proc_e[i][0])
        e_mesh = _mlp(ep_edge, e_mesh, adds=(Gs, Gd), res=e_mesh)
        aggs = _sc_scatter_add(e_mesh, mesh_dst, N_MESH_, 1)
        np_ = P["proc_node_%d" % i]
        W1m, W1a = np_["W1"][:H], np_["W1"][H:]
        if i + 1 < L:
            projs = [proc_e[i + 1][1], proc_e[i + 1][2]]
        else:
            projs = [W1s_m2g]
        outs = _fused_mlp(
            [([m], W1m), (aggs, W1a)], [], np_["b1"], np_["W2"], np_["b2"],
            np_["g"], np_["b"], m, projs, 1024)
        if i + 1 < L:
            m, S, D = outs
        else:
            m, S_m2g = outs

    Gs, Gd = _sc_gather_combine(S_m2g, D_m2g, m2g_src, m2g_dst)
    dp = P["m2g_edge_mlp"]
    dp_edge = dict(dp, W1=W1e_m2g)
    e_m2g = _mlp(dp_edge, e_m2g, adds=(Gs, Gd), res=e_m2g)
    aggs = _sc_scatter_add(e_m2g, m2g_dst, N_GRID_, 5)
    decp = P["dec_node_mlp"]
    W1g, W1a = decp["W1"][:H], decp["W1"][H:]
    g = _fused_mlp(
        [([g], W1g), (aggs, W1a)], [], decp["b1"], decp["W2"], decp["b2"],
        decp["g"], decp["b"], g, [], 1024)

    return _mlp(P["final_mlp"], g, ln=False)


# speedup vs baseline: 2.7245x; 1.2824x over previous
import functools

import jax
import jax.numpy as jnp
from jax import lax
from jax.experimental import pallas as pl
from jax.experimental.pallas import tpu as pltpu
from jax.experimental.pallas import tpu_sc as plsc

H = 128
PREC = lax.Precision.DEFAULT
NC = 2
NS = 16
NW = NC * NS
TB = 128


def _fused_mlp(groups, adds, b1, W2, b2, lng, lnb, res, projs, block_rows):
    n_groups = len(groups)
    xs_counts = [len(xs) for xs, _ in groups]
    n_adds = len(adds)
    has_res = res is not None
    ln = lng is not None
    n_projs = len(projs)

    some_x = groups[0][0][0] if groups else adds[0]
    N = some_x.shape[0]
    dout = W2.shape[1]

    def body(*refs):
        it = iter(refs)
        u = None
        for gi in range(n_groups):
            xs = [next(it)[...] for _ in range(xs_counts[gi])]
            W = next(it)[...]
            x = xs[0]
            for extra in xs[1:]:
                x = x + extra
            t = jnp.dot(x, W, preferred_element_type=jnp.float32,
                        precision=PREC)
            u = t if u is None else u + t
        for _ in range(n_adds):
            a = next(it)[...]
            u = a if u is None else u + a
        b1v = next(it)[...]
        W2v = next(it)[...]
        b2v = next(it)[...]
        u = u + b1v
        h = u * jax.nn.sigmoid(u)
        z = jnp.dot(h, W2v, preferred_element_type=jnp.float32,
                    precision=PREC) + b2v
        if ln:
            gv = next(it)[...]
            bv = next(it)[...]
            mu = jnp.mean(z, axis=-1, keepdims=True)
            zc = z - mu
            var = jnp.mean(zc * zc, axis=-1, keepdims=True)
            z = zc * lax.rsqrt(var + 1e-5) * gv + bv
        if has_res:
            z = next(it)[...] + z
        pws = [next(it)[...] for _ in range(n_projs)]
        outs = list(it)
        outs[0][...] = z
        for k in range(n_projs):
            outs[1 + k][...] = jnp.dot(z, pws[k],
                                       preferred_element_type=jnp.float32,
                                       precision=PREC)

    inputs = []
    in_specs = []

    def add_rowblocked(a):
        inputs.append(a)
        in_specs.append(pl.BlockSpec((block_rows, a.shape[1]),
                                     lambda i: (i, 0)))

    def add_full(a):
        inputs.append(a)
        in_specs.append(pl.BlockSpec(a.shape, lambda i: (0,) * a.ndim))

    for xs, W in groups:
        for x in xs:
            add_rowblocked(x)
        add_full(W)
    for a in adds:
        add_rowblocked(a)
    add_full(b1.reshape(1, -1))
    add_full(W2)
    add_full(b2.reshape(1, -1))
    if ln:
        add_full(lng.reshape(1, -1))
        add_full(lnb.reshape(1, -1))
    if has_res:
        add_rowblocked(res)
    for Pw in projs:
        add_full(Pw)

    out_shapes = [jax.ShapeDtypeStruct((N, dout), jnp.float32)]
    out_shapes += [jax.ShapeDtypeStruct((N, H), jnp.float32)
                   for _ in range(n_projs)]
    out_specs = [pl.BlockSpec((block_rows, dout), lambda i: (i, 0))]
    out_specs += [pl.BlockSpec((block_rows, H), lambda i: (i, 0))
                  for _ in range(n_projs)]

    outs = pl.pallas_call(
        body,
        grid=(pl.cdiv(N, block_rows),),
        in_specs=in_specs,
        out_specs=out_specs,
        out_shape=out_shapes,
    )(*inputs)
    return outs if n_projs else outs[0]


def _mlp(p, x, ln=True, res=None, adds=(), extra_groups=(), projs=(),
         block_rows=1024):
    groups = [([x], p["W1"])] + list(extra_groups)
    lng = p["g"] if ln else None
    lnb = p["b"] if ln else None
    return _fused_mlp(groups, list(adds), p["b1"], p["W2"], p["b2"],
                      lng, lnb, res, list(projs), block_rows)


def _sc_gather_combine(S, D, src, dst):
    E = src.shape[0]
    per_w = E // NW
    nt = per_w // TB
    mesh = plsc.VectorSubcoreMesh(core_axis_name="c", subcore_axis_name="s")

    @functools.partial(
        pl.kernel, mesh=mesh,
        out_type=[jax.ShapeDtypeStruct((E, H), jnp.float32),
                  jax.ShapeDtypeStruct((E, H), jnp.float32)],
        scratch_types=[
            pltpu.VMEM((per_w,), jnp.int32),
            pltpu.VMEM((per_w,), jnp.int32),
            pltpu.VMEM((2, TB, H), jnp.float32),
            pltpu.VMEM((2, TB, H), jnp.float32),
            pltpu.SemaphoreType.DMA,
            [pltpu.SemaphoreType.DMA] * 2,
            [pltpu.SemaphoreType.DMA] * 2,
        ],
    )
    def k(S_hbm, D_hbm, src_hbm, dst_hbm, gs_hbm, gd_hbm,
          sidx, didx, rs, rd, semi, semg, semo):
        wid = lax.axis_index("s") * NC + lax.axis_index("c")
        base = wid * per_w

        c1 = pltpu.async_copy(src_hbm.at[pl.ds(base, per_w)], sidx, semi)
        c2 = pltpu.async_copy(dst_hbm.at[pl.ds(base, per_w)], didx, semi)
        c1.wait()
        c2.wait()

        def issue_gather(t, slot):
            pltpu.async_copy(S_hbm.at[sidx.at[pl.ds(t * TB, TB)]],
                             rs.at[slot], semg[slot])
            pltpu.async_copy(D_hbm.at[didx.at[pl.ds(t * TB, TB)]],
                             rd.at[slot], semg[slot])

        def wait_gather(slot):
            pltpu.make_async_copy(S_hbm.at[pl.ds(0, TB)], rs.at[slot],
                                  semg[slot]).wait()
            pltpu.make_async_copy(D_hbm.at[pl.ds(0, TB)], rd.at[slot],
                                  semg[slot]).wait()

        def issue_store(t, slot):
            off = base + t * TB
            pltpu.async_copy(rs.at[slot], gs_hbm.at[pl.ds(off, TB)],
                             semo[slot])
            pltpu.async_copy(rd.at[slot], gd_hbm.at[pl.ds(off, TB)],
                             semo[slot])

        def wait_store(slot):
            pltpu.make_async_copy(rs.at[slot], gs_hbm.at[pl.ds(0, TB)],
                                  semo[slot]).wait()
            pltpu.make_async_copy(rd.at[slot], gd_hbm.at[pl.ds(0, TB)],
                                  semo[slot]).wait()

        issue_gather(0, 0)

        def pair(i, carry):
            t0 = 2 * i

            @pl.when(i > 0)
            def _():
                wait_store(1)

            issue_gather(t0 + 1, 1)
            wait_gather(0)
            issue_store(t0, 0)

            @pl.when(t0 + 2 < nt)
            def _():
                wait_store(0)
                issue_gather(t0 + 2, 0)

            wait_gather(1)
            issue_store(t0 + 1, 1)
            return carry

        lax.fori_loop(0, nt // 2, pair, 0)
        if nt % 2:
            wait_gather(0)
            issue_store(nt - 1, 0)
            wait_store(0)
            if nt > 1:
                wait_store(1)
        else:
            wait_store(0)
            wait_store(1)

    return k(S, D, src, dst)


def _sc_scatter_add(e, dst, n_nodes, n_chunks):
    E = e.shape[0]
    per_w = E // NW
    nt = per_w // TB
    ch = -(-max(n_nodes + 1, 128) // (n_chunks * 128)) * 128
    stripe = ch // 16
    sp_rows = ch + 16
    mesh = plsc.VectorSubcoreMesh(core_axis_name="c", subcore_axis_name="s")
    zeros = jnp.zeros((ch, H), jnp.float32)

    @functools.partial(
        pl.kernel, mesh=mesh,
        out_type=jax.ShapeDtypeStruct((NC, n_chunks * ch, H), jnp.float32),
        scratch_types=[
            pltpu.VMEM((per_w,), jnp.int32),
            pltpu.VMEM((nt, TB), jnp.int32),
            pltpu.VMEM((2, TB, H), jnp.float32),
            pltpu.VMEM_SHARED((sp_rows, H), jnp.float32),
            pltpu.SemaphoreType.DMA,
            [pltpu.SemaphoreType.DMA] * 2,
            [pltpu.SemaphoreType.DMA] * 2,
        ],
    )
    def k(e_hbm, dst_hbm, z_hbm, out_hbm, didx, lidx, er, acc,
          semi, seme, sema):
        c = lax.axis_index("c")
        s = lax.axis_index("s")
        wid = s * NC + c
        base = wid * per_w

        pltpu.async_copy(dst_hbm.at[pl.ds(base, per_w)], didx, semi).wait()

        def issue_load(t, slot):
            pltpu.async_copy(e_hbm.at[pl.ds(base + t * TB, TB)],
                             er.at[slot], seme[slot])

        def wait_load(slot):
            pltpu.make_async_copy(e_hbm.at[pl.ds(0, TB)], er.at[slot],
                                  seme[slot]).wait()

        def issue_scat(t, slot):
            pltpu.async_copy(er.at[slot], acc.at[lidx.at[t]], sema[slot],
                             add=True)

        def wait_scat(slot):
            pltpu.make_async_copy(er.at[slot], acc.at[pl.ds(0, TB)],
                                  sema[slot]).wait()

        for chunk in range(n_chunks):
            cbase = chunk * ch
            pltpu.sync_copy(z_hbm.at[pl.ds(s * stripe, stripe)],
                            acc.at[pl.ds(s * stripe, stripe)])

            def trans(t, carry):
                for j in range(TB // 16):
                    v = didx[pl.ds(t * TB + j * 16, 16)]
                    inb = (v >= cbase) & (v < cbase + ch)
                    lidx[t, pl.ds(j * 16, 16)] = jnp.where(inb, v - cbase, ch)
                return carry

            lax.fori_loop(0, nt, trans, 0)
            plsc.subcore_barrier()

            issue_load(0, 0)

            def pair(i, carry):
                t0 = 2 * i

                @pl.when(i > 0)
                def _():
                    wait_scat(1)

                issue_load(t0 + 1, 1)
                wait_load(0)
                issue_scat(t0, 0)

                @pl.when(t0 + 2 < nt)
                def _():
                    wait_scat(0)
                    issue_load(t0 + 2, 0)

                wait_load(1)
                issue_scat(t0 + 1, 1)
                return carry

            lax.fori_loop(0, nt // 2, pair, 0)
            if nt % 2:
                wait_load(0)
                issue_scat(nt - 1, 0)
                wait_scat(0)
                if nt > 1:
                    wait_scat(1)
            else:
                wait_scat(0)
                wait_scat(1)
            plsc.subcore_barrier()
            pltpu.sync_copy(
                acc.at[pl.ds(s * stripe, stripe)],
                out_hbm.at[c, pl.ds(cbase + s * stripe, stripe)])
            plsc.subcore_barrier()

    out = k(e, dst, zeros)
    return [out[0, :n_nodes], out[1, :n_nodes]]


def _pad_edges(efeat, src, dst, n_dst, e_pad):
    e = efeat.shape[0]
    pad = e_pad - e
    efeat = jnp.pad(efeat, ((0, pad), (0, 0)))
    src = jnp.pad(src, (0, pad))
    dst = jnp.pad(dst, (0, pad), constant_values=n_dst)
    return efeat, src, dst


def _w1_split(p):
    W1 = p["W1"]
    return W1[:H], W1[H:2 * H], W1[2 * H:]


L = 4
N_MESH_ = 10000
N_GRID_ = 50000


def kernel(grid_nfeat, mesh_nfeat, g2m_efeat, mesh_efeat, m2g_efeat,
           g2m_src, g2m_dst, mesh_src, mesh_dst, m2g_src, m2g_dst, params):
    P = params

    EP_G2M = 204800
    EP_MESH = 163840
    EP_M2G = 155648

    g2m_efeat, g2m_src, g2m_dst = _pad_edges(
        g2m_efeat, g2m_src, g2m_dst, N_MESH_, EP_G2M)
    mesh_efeat, mesh_src, mesh_dst = _pad_edges(
        mesh_efeat, mesh_src, mesh_dst, N_MESH_, EP_MESH)
    m2g_efeat, m2g_src, m2g_dst = _pad_edges(
        m2g_efeat, m2g_src, m2g_dst, N_GRID_, EP_M2G)

    W1e_g2m, W1s_g2m, W1d_g2m = _w1_split(P["g2m_edge_mlp"])
    W1e_m2g, W1s_m2g, W1d_m2g = _w1_split(P["m2g_edge_mlp"])
    proc_e = [_w1_split(P["proc_edge_%d" % i]) for i in range(L)]

    g, S_g2m = _mlp(P["grid_enc"], grid_nfeat, projs=(W1s_g2m,))
    m, D_g2m = _mlp(P["mesh_enc"], mesh_nfeat, projs=(W1d_g2m,))

    def halves(a):
        n = a.shape[0] // 2
        return a[:n], a[n:]

    def edge_stage(e_halves, efeat_halves, enc_p, S, D, src, dst, W1e,
                   edge_p, n_nodes, n_chunks):
        srcs, dsts = halves(src), halves(dst)
        parts = []
        new_e = []
        ep = dict(edge_p, W1=W1e)
        for h in range(2):
            if e_halves is None:
                e_h = _mlp(enc_p, efeat_halves[h])
            else:
                e_h = e_halves[h]
            Gs, Gd = _sc_gather_combine(S, D, srcs[h], dsts[h])
            e_h = _mlp(ep, e_h, adds=(Gs, Gd), res=e_h)
            parts += _sc_scatter_add(e_h, dsts[h], n_nodes, n_chunks)
            new_e.append(e_h)
        return new_e, parts

    _, aggs = edge_stage(None, halves(g2m_efeat), P["g2m_edge_enc"],
                         S_g2m, D_g2m, g2m_src, g2m_dst, W1e_g2m,
                         P["g2m_edge_mlp"], N_MESH_, 1)
    nodep = P["g2m_node_mlp"]
    W1m, W1a = nodep["W1"][:H], nodep["W1"][H:]
    m, S0, D0 = _fused_mlp(
        [([m], W1m), (aggs, W1a)], [], nodep["b1"], nodep["W2"], nodep["b2"],
        nodep["g"], nodep["b"], m, [proc_e[0][1], proc_e[0][2]], 1024)

    g, D_m2g = _mlp(P["enc_grid_mlp"], g, res=g, projs=(W1d_m2g,))

    S, D = S0, D0
    e_mesh = None
    mesh_efeat_h = halves(mesh_efeat)
    for i in range(L):
        e_mesh, aggs = edge_stage(e_mesh, mesh_efeat_h,
                                  P["mesh_edge_enc"], S, D,
                                  mesh_src, mesh_dst, proc_e[i][0],
                                  P["proc_edge_%d" % i], N_MESH_, 1)
        np_ = P["proc_node_%d" % i]
        W1m, W1a = np_["W1"][:H], np_["W1"][H:]
        if i + 1 < L:
            projs = [proc_e[i + 1][1], proc_e[i + 1][2]]
        else:
            projs = [W1s_m2g]
        outs = _fused_mlp(
            [([m], W1m), (aggs, W1a)], [], np_["b1"], np_["W2"], np_["b2"],
            np_["g"], np_["b"], m, projs, 1024)
        if i + 1 < L:
            m, S, D = outs
        else:
            m, S_m2g = outs

    _, aggs = edge_stage(None, halves(m2g_efeat), P["m2g_edge_enc"],
                         S_m2g, D_m2g, m2g_src, m2g_dst, W1e_m2g,
                         P["m2g_edge_mlp"], N_GRID_, 5)
    decp = P["dec_node_mlp"]
    W1g, W1a = decp["W1"][:H], decp["W1"][H:]
    g = _fused_mlp(
        [([g], W1g), (aggs, W1a)], [], decp["b1"], decp["W2"], decp["b2"],
        decp["g"], decp["b"], g, [], 1024)

    return _mlp(P["final_mlp"], g, ln=False)


# speedup vs baseline: 2.7253x; 1.0003x over previous
import functools

import jax
import jax.numpy as jnp
from jax import lax
from jax.experimental import pallas as pl
from jax.experimental.pallas import tpu as pltpu
from jax.experimental.pallas import tpu_sc as plsc

H = 128
PREC = lax.Precision.DEFAULT
NC = 2
NS = 16
NW = NC * NS
TB = 128


def _fused_mlp(groups, adds, b1, W2, b2, lng, lnb, res, projs, block_rows):
    n_groups = len(groups)
    xs_counts = [len(xs) for xs, _ in groups]
    n_adds = len(adds)
    has_res = res is not None
    ln = lng is not None
    n_projs = len(projs)

    some_x = groups[0][0][0] if groups else adds[0]
    N = some_x.shape[0]
    dout = W2.shape[1]

    def body(*refs):
        it = iter(refs)
        u = None
        for gi in range(n_groups):
            xs = [next(it)[...] for _ in range(xs_counts[gi])]
            W = next(it)[...]
            x = xs[0]
            for extra in xs[1:]:
                x = x + extra
            t = jnp.dot(x, W, preferred_element_type=jnp.float32,
                        precision=PREC)
            u = t if u is None else u + t
        for _ in range(n_adds):
            a = next(it)[...]
            u = a if u is None else u + a
        b1v = next(it)[...]
        W2v = next(it)[...]
        b2v = next(it)[...]
        u = u + b1v
        h = u * jax.nn.sigmoid(u)
        z = jnp.dot(h, W2v, preferred_element_type=jnp.float32,
                    precision=PREC) + b2v
        if ln:
            gv = next(it)[...]
            bv = next(it)[...]
            mu = jnp.mean(z, axis=-1, keepdims=True)
            zc = z - mu
            var = jnp.mean(zc * zc, axis=-1, keepdims=True)
            z = zc * lax.rsqrt(var + 1e-5) * gv + bv
        if has_res:
            z = next(it)[...] + z
        pws = [next(it)[...] for _ in range(n_projs)]
        outs = list(it)
        outs[0][...] = z
        for k in range(n_projs):
            outs[1 + k][...] = jnp.dot(z, pws[k],
                                       preferred_element_type=jnp.float32,
                                       precision=PREC)

    inputs = []
    in_specs = []

    def add_rowblocked(a):
        inputs.append(a)
        in_specs.append(pl.BlockSpec((block_rows, a.shape[1]),
                                     lambda i: (i, 0)))

    def add_full(a):
        inputs.append(a)
        in_specs.append(pl.BlockSpec(a.shape, lambda i: (0,) * a.ndim))

    for xs, W in groups:
        for x in xs:
            add_rowblocked(x)
        add_full(W)
    for a in adds:
        add_rowblocked(a)
    add_full(b1.reshape(1, -1))
    add_full(W2)
    add_full(b2.reshape(1, -1))
    if ln:
        add_full(lng.reshape(1, -1))
        add_full(lnb.reshape(1, -1))
    if has_res:
        add_rowblocked(res)
    for Pw in projs:
        add_full(Pw)

    out_shapes = [jax.ShapeDtypeStruct((N, dout), jnp.float32)]
    out_shapes += [jax.ShapeDtypeStruct((N, H), jnp.float32)
                   for _ in range(n_projs)]
    out_specs = [pl.BlockSpec((block_rows, dout), lambda i: (i, 0))]
    out_specs += [pl.BlockSpec((block_rows, H), lambda i: (i, 0))
                  for _ in range(n_projs)]

    outs = pl.pallas_call(
        body,
        grid=(pl.cdiv(N, block_rows),),
        in_specs=in_specs,
        out_specs=out_specs,
        out_shape=out_shapes,
    )(*inputs)
    return outs if n_projs else outs[0]


def _mlp(p, x, ln=True, res=None, adds=(), extra_groups=(), projs=(),
         block_rows=1024):
    groups = [([x], p["W1"])] + list(extra_groups)
    lng = p["g"] if ln else None
    lnb = p["b"] if ln else None
    return _fused_mlp(groups, list(adds), p["b1"], p["W2"], p["b2"],
                      lng, lnb, res, list(projs), block_rows)


def _sc_gather_combine(S, D, src, dst):
    E = src.shape[0]
    per_w = E // NW
    nt = per_w // TB
    mesh = plsc.VectorSubcoreMesh(core_axis_name="c", subcore_axis_name="s")

    @functools.partial(
        pl.kernel, mesh=mesh,
        out_type=[jax.ShapeDtypeStruct((E, H), jnp.float32),
                  jax.ShapeDtypeStruct((E, H), jnp.float32)],
        scratch_types=[
            pltpu.VMEM((per_w,), jnp.int32),
            pltpu.VMEM((per_w,), jnp.int32),
            pltpu.VMEM((2, TB, H), jnp.float32),
            pltpu.VMEM((2, TB, H), jnp.float32),
            pltpu.SemaphoreType.DMA,
            [pltpu.SemaphoreType.DMA] * 2,
            [pltpu.SemaphoreType.DMA] * 2,
        ],
    )
    def k(S_hbm, D_hbm, src_hbm, dst_hbm, gs_hbm, gd_hbm,
          sidx, didx, rs, rd, semi, semg, semo):
        wid = lax.axis_index("s") * NC + lax.axis_index("c")
        base = wid * per_w

        c1 = pltpu.async_copy(src_hbm.at[pl.ds(base, per_w)], sidx, semi)
        c2 = pltpu.async_copy(dst_hbm.at[pl.ds(base, per_w)], didx, semi)
        c1.wait()
        c2.wait()

        def issue_gather(t, slot):
            pltpu.async_copy(S_hbm.at[sidx.at[pl.ds(t * TB, TB)]],
                             rs.at[slot], semg[slot])
            pltpu.async_copy(D_hbm.at[didx.at[pl.ds(t * TB, TB)]],
                             rd.at[slot], semg[slot])

        def wait_gather(slot):
            pltpu.make_async_copy(S_hbm.at[pl.ds(0, TB)], rs.at[slot],
                                  semg[slot]).wait()
            pltpu.make_async_copy(D_hbm.at[pl.ds(0, TB)], rd.at[slot],
                                  semg[slot]).wait()

        def issue_store(t, slot):
            off = base + t * TB
            pltpu.async_copy(rs.at[slot], gs_hbm.at[pl.ds(off, TB)],
                             semo[slot])
            pltpu.async_copy(rd.at[slot], gd_hbm.at[pl.ds(off, TB)],
                             semo[slot])

        def wait_store(slot):
            pltpu.make_async_copy(rs.at[slot], gs_hbm.at[pl.ds(0, TB)],
                                  semo[slot]).wait()
            pltpu.make_async_copy(rd.at[slot], gd_hbm.at[pl.ds(0, TB)],
                                  semo[slot]).wait()

        issue_gather(0, 0)

        def pair(i, carry):
            t0 = 2 * i

            @pl.when(i > 0)
            def _():
                wait_store(1)

            issue_gather(t0 + 1, 1)
            wait_gather(0)
            issue_store(t0, 0)

            @pl.when(t0 + 2 < nt)
            def _():
                wait_store(0)
                issue_gather(t0 + 2, 0)

            wait_gather(1)
            issue_store(t0 + 1, 1)
            return carry

        lax.fori_loop(0, nt // 2, pair, 0)
        if nt % 2:
            wait_gather(0)
            issue_store(nt - 1, 0)
            wait_store(0)
            if nt > 1:
                wait_store(1)
        else:
            wait_store(0)
            wait_store(1)

    return k(S, D, src, dst)


def _sc_scatter_add(e, dst, n_nodes, n_chunks):
    E = e.shape[0]
    per_w = E // NW
    nt = per_w // TB
    ch = -(-max(n_nodes + 1, 128) // (n_chunks * 128)) * 128
    stripe = ch // 16
    sp_rows = ch + 16
    mesh = plsc.VectorSubcoreMesh(core_axis_name="c", subcore_axis_name="s")
    zeros = jnp.zeros((ch, H), jnp.float32)

    @functools.partial(
        pl.kernel, mesh=mesh,
        out_type=jax.ShapeDtypeStruct((NC, n_chunks * ch, H), jnp.float32),
        scratch_types=[
            pltpu.VMEM((per_w,), jnp.int32),
            pltpu.VMEM((nt, TB), jnp.int32),
            pltpu.VMEM((2, TB, H), jnp.float32),
            pltpu.VMEM_SHARED((sp_rows, H), jnp.float32),
            pltpu.SemaphoreType.DMA,
            [pltpu.SemaphoreType.DMA] * 2,
            [pltpu.SemaphoreType.DMA] * 2,
        ],
    )
    def k(e_hbm, dst_hbm, z_hbm, out_hbm, didx, lidx, er, acc,
          semi, seme, sema):
        c = lax.axis_index("c")
        s = lax.axis_index("s")
        wid = s * NC + c
        base = wid * per_w

        pltpu.async_copy(dst_hbm.at[pl.ds(base, per_w)], didx, semi).wait()

        def issue_load(t, slot):
            pltpu.async_copy(e_hbm.at[pl.ds(base + t * TB, TB)],
                             er.at[slot], seme[slot])

        def wait_load(slot):
            pltpu.make_async_copy(e_hbm.at[pl.ds(0, TB)], er.at[slot],
                                  seme[slot]).wait()

        def issue_scat(t, slot):
            if n_chunks == 1:
                rows = didx.at[pl.ds(t * TB, TB)]
            else:
                rows = lidx.at[t]
            pltpu.async_copy(er.at[slot], acc.at[rows], sema[slot],
                             add=True)

        def wait_scat(slot):
            pltpu.make_async_copy(er.at[slot], acc.at[pl.ds(0, TB)],
                                  sema[slot]).wait()

        for chunk in range(n_chunks):
            cbase = chunk * ch
            pltpu.sync_copy(z_hbm.at[pl.ds(s * stripe, stripe)],
                            acc.at[pl.ds(s * stripe, stripe)])

            if n_chunks > 1:
                def trans(t, carry):
                    for j in range(TB // 16):
                        v = didx[pl.ds(t * TB + j * 16, 16)]
                        inb = (v >= cbase) & (v < cbase + ch)
                        lidx[t, pl.ds(j * 16, 16)] = jnp.where(
                            inb, v - cbase, ch)
                    return carry

                lax.fori_loop(0, nt, trans, 0)
            plsc.subcore_barrier()

            issue_load(0, 0)

            def pair(i, carry):
                t0 = 2 * i

                @pl.when(i > 0)
                def _():
                    wait_scat(1)

                issue_load(t0 + 1, 1)
                wait_load(0)
                issue_scat(t0, 0)

                @pl.when(t0 + 2 < nt)
                def _():
                    wait_scat(0)
                    issue_load(t0 + 2, 0)

                wait_load(1)
                issue_scat(t0 + 1, 1)
                return carry

            lax.fori_loop(0, nt // 2, pair, 0)
            if nt % 2:
                wait_load(0)
                issue_scat(nt - 1, 0)
                wait_scat(0)
                if nt > 1:
                    wait_scat(1)
            else:
                wait_scat(0)
                wait_scat(1)
            plsc.subcore_barrier()
            pltpu.sync_copy(
                acc.at[pl.ds(s * stripe, stripe)],
                out_hbm.at[c, pl.ds(cbase + s * stripe, stripe)])
            plsc.subcore_barrier()

    out = k(e, dst, zeros)
    return [out[0, :n_nodes], out[1, :n_nodes]]


def _pad_edges(efeat, src, dst, n_dst, e_pad):
    e = efeat.shape[0]
    pad = e_pad - e
    efeat = jnp.pad(efeat, ((0, pad), (0, 0)))
    src = jnp.pad(src, (0, pad))
    dst = jnp.pad(dst, (0, pad), constant_values=n_dst)
    return efeat, src, dst


def _w1_split(p):
    W1 = p["W1"]
    return W1[:H], W1[H:2 * H], W1[2 * H:]


L = 4
N_MESH_ = 10000
N_GRID_ = 50000


def kernel(grid_nfeat, mesh_nfeat, g2m_efeat, mesh_efeat, m2g_efeat,
           g2m_src, g2m_dst, mesh_src, mesh_dst, m2g_src, m2g_dst, params):
    P = params

    EP_G2M = 204800
    EP_MESH = 163840
    EP_M2G = 155648

    g2m_efeat, g2m_src, g2m_dst = _pad_edges(
        g2m_efeat, g2m_src, g2m_dst, N_MESH_, EP_G2M)
    mesh_efeat, mesh_src, mesh_dst = _pad_edges(
        mesh_efeat, mesh_src, mesh_dst, N_MESH_, EP_MESH)
    m2g_efeat, m2g_src, m2g_dst = _pad_edges(
        m2g_efeat, m2g_src, m2g_dst, N_GRID_, EP_M2G)

    W1e_g2m, W1s_g2m, W1d_g2m = _w1_split(P["g2m_edge_mlp"])
    W1e_m2g, W1s_m2g, W1d_m2g = _w1_split(P["m2g_edge_mlp"])
    proc_e = [_w1_split(P["proc_edge_%d" % i]) for i in range(L)]

    g, S_g2m = _mlp(P["grid_enc"], grid_nfeat, projs=(W1s_g2m,))
    m, D_g2m = _mlp(P["mesh_enc"], mesh_nfeat, projs=(W1d_g2m,))

    def halves(a):
        n = a.shape[0] // 2
        return a[:n], a[n:]

    def edge_stage(e_halves, efeat_halves, enc_p, S, D, src, dst, W1e,
                   edge_p, n_nodes, n_chunks):
        srcs, dsts = halves(src), halves(dst)
        parts = []
        new_e = []
        ep = dict(edge_p, W1=W1e)
        for h in range(2):
            if e_halves is None:
                e_h = _mlp(enc_p, efeat_halves[h])
            else:
                e_h = e_halves[h]
            Gs, Gd = _sc_gather_combine(S, D, srcs[h], dsts[h])
            e_h = _mlp(ep, e_h, adds=(Gs, Gd), res=e_h)
            parts += _sc_scatter_add(e_h, dsts[h], n_nodes, n_chunks)
            new_e.append(e_h)
        return new_e, parts

    _, aggs = edge_stage(None, halves(g2m_efeat), P["g2m_edge_enc"],
                         S_g2m, D_g2m, g2m_src, g2m_dst, W1e_g2m,
                         P["g2m_edge_mlp"], N_MESH_, 1)
    nodep = P["g2m_node_mlp"]
    W1m, W1a = nodep["W1"][:H], nodep["W1"][H:]
    m, S0, D0 = _fused_mlp(
        [([m], W1m), (aggs, W1a)], [], nodep["b1"], nodep["W2"], nodep["b2"],
        nodep["g"], nodep["b"], m, [proc_e[0][1], proc_e[0][2]], 1024)

    g, D_m2g = _mlp(P["enc_grid_mlp"], g, res=g, projs=(W1d_m2g,))

    S, D = S0, D0
    e_mesh = None
    mesh_efeat_h = halves(mesh_efeat)
    for i in range(L):
        e_mesh, aggs = edge_stage(e_mesh, mesh_efeat_h,
                                  P["mesh_edge_enc"], S, D,
                                  mesh_src, mesh_dst, proc_e[i][0],
                                  P["proc_edge_%d" % i], N_MESH_, 1)
        np_ = P["proc_node_%d" % i]
        W1m, W1a = np_["W1"][:H], np_["W1"][H:]
        if i + 1 < L:
            projs = [proc_e[i + 1][1], proc_e[i + 1][2]]
        else:
            projs = [W1s_m2g]
        outs = _fused_mlp(
            [([m], W1m), (aggs, W1a)], [], np_["b1"], np_["W2"], np_["b2"],
            np_["g"], np_["b"], m, projs, 1024)
        if i + 1 < L:
            m, S, D = outs
        else:
            m, S_m2g = outs

    _, aggs = edge_stage(None, halves(m2g_efeat), P["m2g_edge_enc"],
                         S_m2g, D_m2g, m2g_src, m2g_dst, W1e_m2g,
                         P["m2g_edge_mlp"], N_GRID_, 5)
    decp = P["dec_node_mlp"]
    W1g, W1a = decp["W1"][:H], decp["W1"][H:]
    g = _fused_mlp(
        [([g], W1g), (aggs, W1a)], [], decp["b1"], decp["W2"], decp["b2"],
        decp["g"], decp["b"], g, [], 1024)

    return _mlp(P["final_mlp"], g, ln=False)


# speedup vs baseline: 2.8821x; 1.0575x over previous
import functools

import jax
import jax.numpy as jnp
from jax import lax
from jax.experimental import pallas as pl
from jax.experimental.pallas import tpu as pltpu
from jax.experimental.pallas import tpu_sc as plsc

H = 128
PREC = lax.Precision.DEFAULT
NC = 2
NS = 16
NW = NC * NS
TB = 128


def _fused_mlp(groups, adds, b1, W2, b2, lng, lnb, res, projs, block_rows):
    n_groups = len(groups)
    xs_counts = [len(xs) for xs, _ in groups]
    n_adds = len(adds)
    has_res = res is not None
    ln = lng is not None
    n_projs = len(projs)

    some_x = groups[0][0][0] if groups else adds[0]
    N = some_x.shape[0]
    dout = W2.shape[1]

    def body(*refs):
        it = iter(refs)
        u = None
        for gi in range(n_groups):
            xs = [next(it)[...] for _ in range(xs_counts[gi])]
            W = next(it)[...]
            x = xs[0]
            for extra in xs[1:]:
                x = x + extra
            t = jnp.dot(x, W, preferred_element_type=jnp.float32,
                        precision=PREC)
            u = t if u is None else u + t
        for _ in range(n_adds):
            a = next(it)[...]
            u = a if u is None else u + a
        b1v = next(it)[...]
        W2v = next(it)[...]
        b2v = next(it)[...]
        u = u + b1v
        h = u * jax.nn.sigmoid(u)
        z = jnp.dot(h, W2v, preferred_element_type=jnp.float32,
                    precision=PREC) + b2v
        if ln:
            gv = next(it)[...]
            bv = next(it)[...]
            mu = jnp.mean(z, axis=-1, keepdims=True)
            zc = z - mu
            var = jnp.mean(zc * zc, axis=-1, keepdims=True)
            z = zc * lax.rsqrt(var + 1e-5) * gv + bv
        if has_res:
            z = next(it)[...] + z
        pws = [next(it)[...] for _ in range(n_projs)]
        outs = list(it)
        outs[0][...] = z
        for k in range(n_projs):
            outs[1 + k][...] = jnp.dot(z, pws[k],
                                       preferred_element_type=jnp.float32,
                                       precision=PREC)

    inputs = []
    in_specs = []

    def add_rowblocked(a):
        inputs.append(a)
        in_specs.append(pl.BlockSpec((block_rows, a.shape[1]),
                                     lambda i: (i, 0)))

    def add_full(a):
        inputs.append(a)
        in_specs.append(pl.BlockSpec(a.shape, lambda i: (0,) * a.ndim))

    for xs, W in groups:
        for x in xs:
            add_rowblocked(x)
        add_full(W)
    for a in adds:
        add_rowblocked(a)
    add_full(b1.reshape(1, -1))
    add_full(W2)
    add_full(b2.reshape(1, -1))
    if ln:
        add_full(lng.reshape(1, -1))
        add_full(lnb.reshape(1, -1))
    if has_res:
        add_rowblocked(res)
    for Pw in projs:
        add_full(Pw)

    out_shapes = [jax.ShapeDtypeStruct((N, dout), jnp.float32)]
    out_shapes += [jax.ShapeDtypeStruct((N, H), jnp.float32)
                   for _ in range(n_projs)]
    out_specs = [pl.BlockSpec((block_rows, dout), lambda i: (i, 0))]
    out_specs += [pl.BlockSpec((block_rows, H), lambda i: (i, 0))
                  for _ in range(n_projs)]

    outs = pl.pallas_call(
        body,
        grid=(pl.cdiv(N, block_rows),),
        in_specs=in_specs,
        out_specs=out_specs,
        out_shape=out_shapes,
    )(*inputs)
    return outs if n_projs else outs[0]


def _mlp(p, x, ln=True, res=None, adds=(), extra_groups=(), projs=(),
         block_rows=1024):
    groups = [([x], p["W1"])] + list(extra_groups)
    lng = p["g"] if ln else None
    lnb = p["b"] if ln else None
    return _fused_mlp(groups, list(adds), p["b1"], p["W2"], p["b2"],
                      lng, lnb, res, list(projs), block_rows)


def _sc_gather_combine(S, D, src, dst):
    E = src.shape[0]
    per_w = E // NW
    nt = per_w // TB
    mesh = plsc.VectorSubcoreMesh(core_axis_name="c", subcore_axis_name="s")

    @functools.partial(
        pl.kernel, mesh=mesh,
        out_type=jax.ShapeDtypeStruct((E, H), jnp.float32),
        scratch_types=[
            pltpu.VMEM((per_w,), jnp.int32),
            pltpu.VMEM((per_w,), jnp.int32),
            pltpu.VMEM((2, TB, H), jnp.float32),
            pltpu.SemaphoreType.DMA,
            [pltpu.SemaphoreType.DMA] * 2,
            [pltpu.SemaphoreType.DMA] * 2,
            [pltpu.SemaphoreType.DMA] * 2,
        ],
    )
    def k(S_hbm, D_hbm, src_hbm, dst_hbm, g_hbm,
          sidx, didx, rs, semi, semg, sema, semo):
        wid = lax.axis_index("s") * NC + lax.axis_index("c")
        base = wid * per_w

        c1 = pltpu.async_copy(src_hbm.at[pl.ds(base, per_w)], sidx, semi)
        c2 = pltpu.async_copy(dst_hbm.at[pl.ds(base, per_w)], didx, semi)
        c1.wait()
        c2.wait()

        def issue_gather(t, slot):
            pltpu.async_copy(S_hbm.at[sidx.at[pl.ds(t * TB, TB)]],
                             rs.at[slot], semg[slot])

        def wait_gather(slot):
            pltpu.make_async_copy(S_hbm.at[pl.ds(0, TB)], rs.at[slot],
                                  semg[slot]).wait()

        def issue_store(t, slot):
            pltpu.async_copy(D_hbm.at[didx.at[pl.ds(t * TB, TB)]],
                             rs.at[slot], sema[slot], add=True)
            pltpu.make_async_copy(D_hbm.at[pl.ds(0, TB)], rs.at[slot],
                                  sema[slot]).wait()
            pltpu.async_copy(rs.at[slot], g_hbm.at[pl.ds(base + t * TB, TB)],
                             semo[slot])

        def wait_store(slot):
            pltpu.make_async_copy(rs.at[slot], g_hbm.at[pl.ds(0, TB)],
                                  semo[slot]).wait()

        issue_gather(0, 0)

        def pair(i, carry):
            t0 = 2 * i

            @pl.when(i > 0)
            def _():
                wait_store(1)

            issue_gather(t0 + 1, 1)
            wait_gather(0)
            issue_store(t0, 0)

            @pl.when(t0 + 2 < nt)
            def _():
                wait_store(0)
                issue_gather(t0 + 2, 0)

            wait_gather(1)
            issue_store(t0 + 1, 1)
            return carry

        lax.fori_loop(0, nt // 2, pair, 0)
        if nt % 2:
            wait_gather(0)
            issue_store(nt - 1, 0)
            wait_store(0)
            if nt > 1:
                wait_store(1)
        else:
            wait_store(0)
            wait_store(1)

    return k(S, D, src, dst)


def _sc_scatter_add(e, dst, n_nodes, n_chunks):
    E = e.shape[0]
    per_w = E // NW
    nt = per_w // TB
    ch = -(-max(n_nodes + 1, 128) // (n_chunks * 128)) * 128
    stripe = ch // 16
    sp_rows = ch + 16
    mesh = plsc.VectorSubcoreMesh(core_axis_name="c", subcore_axis_name="s")
    zeros = jnp.zeros((ch, H), jnp.float32)

    @functools.partial(
        pl.kernel, mesh=mesh,
        out_type=jax.ShapeDtypeStruct((NC, n_chunks * ch, H), jnp.float32),
        scratch_types=[
            pltpu.VMEM((per_w,), jnp.int32),
            pltpu.VMEM((nt, TB), jnp.int32),
            pltpu.VMEM((2, TB, H), jnp.float32),
            pltpu.VMEM_SHARED((sp_rows, H), jnp.float32),
            pltpu.SemaphoreType.DMA,
            [pltpu.SemaphoreType.DMA] * 2,
            [pltpu.SemaphoreType.DMA] * 2,
        ],
    )
    def k(e_hbm, dst_hbm, z_hbm, out_hbm, didx, lidx, er, acc,
          semi, seme, sema):
        c = lax.axis_index("c")
        s = lax.axis_index("s")
        wid = s * NC + c
        base = wid * per_w

        pltpu.async_copy(dst_hbm.at[pl.ds(base, per_w)], didx, semi).wait()

        def issue_load(t, slot):
            pltpu.async_copy(e_hbm.at[pl.ds(base + t * TB, TB)],
                             er.at[slot], seme[slot])

        def wait_load(slot):
            pltpu.make_async_copy(e_hbm.at[pl.ds(0, TB)], er.at[slot],
                                  seme[slot]).wait()

        def issue_scat(t, slot):
            if n_chunks == 1:
                rows = didx.at[pl.ds(t * TB, TB)]
            else:
                rows = lidx.at[t]
            pltpu.async_copy(er.at[slot], acc.at[rows], sema[slot],
                             add=True)

        def wait_scat(slot):
            pltpu.make_async_copy(er.at[slot], acc.at[pl.ds(0, TB)],
                                  sema[slot]).wait()

        for chunk in range(n_chunks):
            cbase = chunk * ch
            pltpu.sync_copy(z_hbm.at[pl.ds(s * stripe, stripe)],
                            acc.at[pl.ds(s * stripe, stripe)])

            if n_chunks > 1:
                def trans(t, carry):
                    for j in range(TB // 16):
                        v = didx[pl.ds(t * TB + j * 16, 16)]
                        inb = (v >= cbase) & (v < cbase + ch)
                        lidx[t, pl.ds(j * 16, 16)] = jnp.where(
                            inb, v - cbase, ch)
                    return carry

                lax.fori_loop(0, nt, trans, 0)
            plsc.subcore_barrier()

            issue_load(0, 0)

            def pair(i, carry):
                t0 = 2 * i

                @pl.when(i > 0)
                def _():
                    wait_scat(1)

                issue_load(t0 + 1, 1)
                wait_load(0)
                issue_scat(t0, 0)

                @pl.when(t0 + 2 < nt)
                def _():
                    wait_scat(0)
                    issue_load(t0 + 2, 0)

                wait_load(1)
                issue_scat(t0 + 1, 1)
                return carry

            lax.fori_loop(0, nt // 2, pair, 0)
            if nt % 2:
                wait_load(0)
                issue_scat(nt - 1, 0)
                wait_scat(0)
                if nt > 1:
                    wait_scat(1)
            else:
                wait_scat(0)
                wait_scat(1)
            plsc.subcore_barrier()
            pltpu.sync_copy(
                acc.at[pl.ds(s * stripe, stripe)],
                out_hbm.at[c, pl.ds(cbase + s * stripe, stripe)])
            plsc.subcore_barrier()

    out = k(e, dst, zeros)
    return [out[0, :n_nodes], out[1, :n_nodes]]


def _pad_edges(efeat, src, dst, n_dst, e_pad):
    e = efeat.shape[0]
    pad = e_pad - e
    efeat = jnp.pad(efeat, ((0, pad), (0, 0)))
    src = jnp.pad(src, (0, pad))
    dst = jnp.pad(dst, (0, pad), constant_values=n_dst)
    return efeat, src, dst


def _w1_split(p):
    W1 = p["W1"]
    return W1[:H], W1[H:2 * H], W1[2 * H:]


L = 4
N_MESH_ = 10000
N_GRID_ = 50000


def kernel(grid_nfeat, mesh_nfeat, g2m_efeat, mesh_efeat, m2g_efeat,
           g2m_src, g2m_dst, mesh_src, mesh_dst, m2g_src, m2g_dst, params):
    P = params

    EP_G2M = 204800
    EP_MESH = 163840
    EP_M2G = 155648

    g2m_efeat, g2m_src, g2m_dst = _pad_edges(
        g2m_efeat, g2m_src, g2m_dst, N_MESH_, EP_G2M)
    mesh_efeat, mesh_src, mesh_dst = _pad_edges(
        mesh_efeat, mesh_src, mesh_dst, N_MESH_, EP_MESH)
    m2g_efeat, m2g_src, m2g_dst = _pad_edges(
        m2g_efeat, m2g_src, m2g_dst, N_GRID_, EP_M2G)

    W1e_g2m, W1s_g2m, W1d_g2m = _w1_split(P["g2m_edge_mlp"])
    W1e_m2g, W1s_m2g, W1d_m2g = _w1_split(P["m2g_edge_mlp"])
    proc_e = [_w1_split(P["proc_edge_%d" % i]) for i in range(L)]

    g, S_g2m = _mlp(P["grid_enc"], grid_nfeat, projs=(W1s_g2m,))
    m, D_g2m = _mlp(P["mesh_enc"], mesh_nfeat, projs=(W1d_g2m,))

    def halves(a):
        n = a.shape[0] // 2
        return a[:n], a[n:]

    def edge_stage(e_halves, efeat_halves, enc_p, S, D, src, dst, W1e,
                   edge_p, n_nodes, n_chunks):
        srcs, dsts = halves(src), halves(dst)
        parts = []
        new_e = []
        ep = dict(edge_p, W1=W1e)
        for h in range(2):
            if e_halves is None:
                e_h = _mlp(enc_p, efeat_halves[h])
            else:
                e_h = e_halves[h]
            G = _sc_gather_combine(S, D, srcs[h], dsts[h])
            e_h = _mlp(ep, e_h, adds=(G,), res=e_h)
            parts += _sc_scatter_add(e_h, dsts[h], n_nodes, n_chunks)
            new_e.append(e_h)
        return new_e, parts

    _, aggs = edge_stage(None, halves(g2m_efeat), P["g2m_edge_enc"],
                         S_g2m, D_g2m, g2m_src, g2m_dst, W1e_g2m,
                         P["g2m_edge_mlp"], N_MESH_, 1)
    nodep = P["g2m_node_mlp"]
    W1m, W1a = nodep["W1"][:H], nodep["W1"][H:]
    m, S0, D0 = _fused_mlp(
        [([m], W1m), (aggs, W1a)], [], nodep["b1"], nodep["W2"], nodep["b2"],
        nodep["g"], nodep["b"], m, [proc_e[0][1], proc_e[0][2]], 1024)

    g, D_m2g = _mlp(P["enc_grid_mlp"], g, res=g, projs=(W1d_m2g,))

    S, D = S0, D0
    e_mesh = None
    mesh_efeat_h = halves(mesh_efeat)
    for i in range(L):
        e_mesh, aggs = edge_stage(e_mesh, mesh_efeat_h,
                                  P["mesh_edge_enc"], S, D,
                                  mesh_src, mesh_dst, proc_e[i][0],
                                  P["proc_edge_%d" % i], N_MESH_, 1)
        np_ = P["proc_node_%d" % i]
        W1m, W1a = np_["W1"][:H], np_["W1"][H:]
        if i + 1 < L:
            projs = [proc_e[i + 1][1], proc_e[i + 1][2]]
        else:
            projs = [W1s_m2g]
        outs = _fused_mlp(
            [([m], W1m), (aggs, W1a)], [], np_["b1"], np_["W2"], np_["b2"],
            np_["g"], np_["b"], m, projs, 1024)
        if i + 1 < L:
            m, S, D = outs
        else:
            m, S_m2g = outs

    _, aggs = edge_stage(None, halves(m2g_efeat), P["m2g_edge_enc"],
                         S_m2g, D_m2g, m2g_src, m2g_dst, W1e_m2g,
                         P["m2g_edge_mlp"], N_GRID_, 5)
    decp = P["dec_node_mlp"]
    W1g, W1a = decp["W1"][:H], decp["W1"][H:]
    g = _fused_mlp(
        [([g], W1g), (aggs, W1a)], [], decp["b1"], decp["W2"], decp["b2"],
        decp["g"], decp["b"], g, [], 1024)

    return _mlp(P["final_mlp"], g, ln=False)


# speedup vs baseline: 2.9272x; 1.0157x over previous
import functools

import jax
import jax.numpy as jnp
from jax import lax
from jax.experimental import pallas as pl
from jax.experimental.pallas import tpu as pltpu
from jax.experimental.pallas import tpu_sc as plsc

H = 128
PREC = lax.Precision.DEFAULT
NC = 2
NS = 16
NW = NC * NS
TB = 128


def _fused_mlp(groups, adds, b1, W2, b2, lng, lnb, res, projs, block_rows):
    n_groups = len(groups)
    xs_counts = [len(xs) for xs, _ in groups]
    n_adds = len(adds)
    has_res = res is not None
    ln = lng is not None
    n_projs = len(projs)

    some_x = groups[0][0][0] if groups else adds[0]
    N = some_x.shape[0]
    dout = W2.shape[1]

    def body(*refs):
        it = iter(refs)
        u = None
        for gi in range(n_groups):
            xs = [next(it)[...] for _ in range(xs_counts[gi])]
            W = next(it)[...]
            x = xs[0]
            for extra in xs[1:]:
                x = x + extra
            t = jnp.dot(x, W, preferred_element_type=jnp.float32,
                        precision=PREC)
            u = t if u is None else u + t
        for _ in range(n_adds):
            a = next(it)[...]
            u = a if u is None else u + a
        b1v = next(it)[...]
        W2v = next(it)[...]
        b2v = next(it)[...]
        u = u + b1v
        h = u * jax.nn.sigmoid(u)
        z = jnp.dot(h, W2v, preferred_element_type=jnp.float32,
                    precision=PREC) + b2v
        if ln:
            gv = next(it)[...]
            bv = next(it)[...]
            mu = jnp.mean(z, axis=-1, keepdims=True)
            zc = z - mu
            var = jnp.mean(zc * zc, axis=-1, keepdims=True)
            z = zc * lax.rsqrt(var + 1e-5) * gv + bv
        if has_res:
            z = next(it)[...] + z
        pws = [next(it)[...] for _ in range(n_projs)]
        outs = list(it)
        outs[0][...] = z
        for k in range(n_projs):
            outs[1 + k][...] = jnp.dot(z, pws[k],
                                       preferred_element_type=jnp.float32,
                                       precision=PREC)

    inputs = []
    in_specs = []

    def add_rowblocked(a):
        inputs.append(a)
        in_specs.append(pl.BlockSpec((block_rows, a.shape[1]),
                                     lambda i: (i, 0)))

    def add_full(a):
        inputs.append(a)
        in_specs.append(pl.BlockSpec(a.shape, lambda i: (0,) * a.ndim))

    for xs, W in groups:
        for x in xs:
            add_rowblocked(x)
        add_full(W)
    for a in adds:
        add_rowblocked(a)
    add_full(b1.reshape(1, -1))
    add_full(W2)
    add_full(b2.reshape(1, -1))
    if ln:
        add_full(lng.reshape(1, -1))
        add_full(lnb.reshape(1, -1))
    if has_res:
        add_rowblocked(res)
    for Pw in projs:
        add_full(Pw)

    out_shapes = [jax.ShapeDtypeStruct((N, dout), jnp.float32)]
    out_shapes += [jax.ShapeDtypeStruct((N, H), jnp.float32)
                   for _ in range(n_projs)]
    out_specs = [pl.BlockSpec((block_rows, dout), lambda i: (i, 0))]
    out_specs += [pl.BlockSpec((block_rows, H), lambda i: (i, 0))
                  for _ in range(n_projs)]

    outs = pl.pallas_call(
        body,
        grid=(pl.cdiv(N, block_rows),),
        in_specs=in_specs,
        out_specs=out_specs,
        out_shape=out_shapes,
    )(*inputs)
    return outs if n_projs else outs[0]


def _mlp(p, x, ln=True, res=None, adds=(), extra_groups=(), projs=(),
         block_rows=1024):
    groups = [([x], p["W1"])] + list(extra_groups)
    lng = p["g"] if ln else None
    lnb = p["b"] if ln else None
    return _fused_mlp(groups, list(adds), p["b1"], p["W2"], p["b2"],
                      lng, lnb, res, list(projs), block_rows)


def _sc_gather_combine(S, D, src, dst):
    E = src.shape[0]
    per_w = E // NW
    nt = per_w // TB
    mesh = plsc.VectorSubcoreMesh(core_axis_name="c", subcore_axis_name="s")

    @functools.partial(
        pl.kernel, mesh=mesh,
        out_type=jax.ShapeDtypeStruct((E, H), jnp.float32),
        scratch_types=[
            pltpu.VMEM((per_w,), jnp.int32),
            pltpu.VMEM((per_w,), jnp.int32),
            pltpu.VMEM((2, TB, H), jnp.float32),
            pltpu.SemaphoreType.DMA,
            [pltpu.SemaphoreType.DMA] * 2,
            [pltpu.SemaphoreType.DMA] * 2,
            [pltpu.SemaphoreType.DMA] * 2,
        ],
    )
    def k(S_hbm, D_hbm, src_hbm, dst_hbm, g_hbm,
          sidx, didx, rs, semi, semg, sema, semo):
        wid = lax.axis_index("s") * NC + lax.axis_index("c")
        base = wid * per_w

        c1 = pltpu.async_copy(src_hbm.at[pl.ds(base, per_w)], sidx, semi)
        c2 = pltpu.async_copy(dst_hbm.at[pl.ds(base, per_w)], didx, semi)
        c1.wait()
        c2.wait()

        def issue_gather(t, slot):
            pltpu.async_copy(S_hbm.at[sidx.at[pl.ds(t * TB, TB)]],
                             rs.at[slot], semg[slot])

        def wait_gather(slot):
            pltpu.make_async_copy(S_hbm.at[pl.ds(0, TB)], rs.at[slot],
                                  semg[slot]).wait()

        def issue_store(t, slot):
            pltpu.async_copy(D_hbm.at[didx.at[pl.ds(t * TB, TB)]],
                             rs.at[slot], sema[slot], add=True)
            pltpu.make_async_copy(D_hbm.at[pl.ds(0, TB)], rs.at[slot],
                                  sema[slot]).wait()
            pltpu.async_copy(rs.at[slot], g_hbm.at[pl.ds(base + t * TB, TB)],
                             semo[slot])

        def wait_store(slot):
            pltpu.make_async_copy(rs.at[slot], g_hbm.at[pl.ds(0, TB)],
                                  semo[slot]).wait()

        issue_gather(0, 0)

        def pair(i, carry):
            t0 = 2 * i

            @pl.when(i > 0)
            def _():
                wait_store(1)

            issue_gather(t0 + 1, 1)
            wait_gather(0)
            issue_store(t0, 0)

            @pl.when(t0 + 2 < nt)
            def _():
                wait_store(0)
                issue_gather(t0 + 2, 0)

            wait_gather(1)
            issue_store(t0 + 1, 1)
            return carry

        lax.fori_loop(0, nt // 2, pair, 0)
        if nt % 2:
            wait_gather(0)
            issue_store(nt - 1, 0)
            wait_store(0)
            if nt > 1:
                wait_store(1)
        else:
            wait_store(0)
            wait_store(1)

    return k(S, D, src, dst)


def _sc_scatter_add(e, dst, n_nodes, n_chunks, tb=TB):
    E = e.shape[0]
    per_w = E // NW
    nt = per_w // tb
    ch = -(-max(n_nodes + 1, 128) // (n_chunks * 128)) * 128
    stripe = ch // 16
    sp_rows = ch + 16
    mesh = plsc.VectorSubcoreMesh(core_axis_name="c", subcore_axis_name="s")
    zeros = jnp.zeros((ch, H), jnp.float32)

    @functools.partial(
        pl.kernel, mesh=mesh,
        out_type=jax.ShapeDtypeStruct((NC, n_chunks * ch, H), jnp.float32),
        scratch_types=[
            pltpu.VMEM((per_w,), jnp.int32),
            pltpu.VMEM((nt, tb), jnp.int32),
            pltpu.VMEM((2, tb, H), jnp.float32),
            pltpu.VMEM_SHARED((sp_rows, H), jnp.float32),
            pltpu.SemaphoreType.DMA,
            [pltpu.SemaphoreType.DMA] * 2,
            [pltpu.SemaphoreType.DMA] * 2,
        ],
    )
    def k(e_hbm, dst_hbm, z_hbm, out_hbm, didx, lidx, er, acc,
          semi, seme, sema):
        c = lax.axis_index("c")
        s = lax.axis_index("s")
        wid = s * NC + c
        base = wid * per_w

        pltpu.async_copy(dst_hbm.at[pl.ds(base, per_w)], didx, semi).wait()

        def issue_load(t, slot):
            pltpu.async_copy(e_hbm.at[pl.ds(base + t * tb, tb)],
                             er.at[slot], seme[slot])

        def wait_load(slot):
            pltpu.make_async_copy(e_hbm.at[pl.ds(0, tb)], er.at[slot],
                                  seme[slot]).wait()

        def issue_scat(t, slot):
            if n_chunks == 1:
                rows = didx.at[pl.ds(t * tb, tb)]
            else:
                rows = lidx.at[t]
            pltpu.async_copy(er.at[slot], acc.at[rows], sema[slot],
                             add=True)

        def wait_scat(slot):
            pltpu.make_async_copy(er.at[slot], acc.at[pl.ds(0, tb)],
                                  sema[slot]).wait()

        for chunk in range(n_chunks):
            cbase = chunk * ch
            pltpu.sync_copy(z_hbm.at[pl.ds(s * stripe, stripe)],
                            acc.at[pl.ds(s * stripe, stripe)])

            if n_chunks > 1:
                def trans(t, carry):
                    for j in range(tb // 16):
                        v = didx[pl.ds(t * tb + j * 16, 16)]
                        inb = (v >= cbase) & (v < cbase + ch)
                        lidx[t, pl.ds(j * 16, 16)] = jnp.where(
                            inb, v - cbase, ch)
                    return carry

                lax.fori_loop(0, nt, trans, 0)
            plsc.subcore_barrier()

            issue_load(0, 0)

            def pair(i, carry):
                t0 = 2 * i

                @pl.when(i > 0)
                def _():
                    wait_scat(1)

                issue_load(t0 + 1, 1)
                wait_load(0)
                issue_scat(t0, 0)

                @pl.when(t0 + 2 < nt)
                def _():
                    wait_scat(0)
                    issue_load(t0 + 2, 0)

                wait_load(1)
                issue_scat(t0 + 1, 1)
                return carry

            lax.fori_loop(0, nt // 2, pair, 0)
            if nt % 2:
                wait_load(0)
                issue_scat(nt - 1, 0)
                wait_scat(0)
                if nt > 1:
                    wait_scat(1)
            else:
                wait_scat(0)
                wait_scat(1)
            plsc.subcore_barrier()
            pltpu.sync_copy(
                acc.at[pl.ds(s * stripe, stripe)],
                out_hbm.at[c, pl.ds(cbase + s * stripe, stripe)])
            plsc.subcore_barrier()

    out = k(e, dst, zeros)
    return [out[0, :n_nodes], out[1, :n_nodes]]


def _pad_edges(efeat, src, dst, n_dst, e_pad):
    e = efeat.shape[0]
    pad = e_pad - e
    efeat = jnp.pad(efeat, ((0, pad), (0, 0)))
    src = jnp.pad(src, (0, pad))
    dst = jnp.pad(dst, (0, pad), constant_values=n_dst)
    return efeat, src, dst


def _w1_split(p):
    W1 = p["W1"]
    return W1[:H], W1[H:2 * H], W1[2 * H:]


L = 4
N_MESH_ = 10000
N_GRID_ = 50000


def kernel(grid_nfeat, mesh_nfeat, g2m_efeat, mesh_efeat, m2g_efeat,
           g2m_src, g2m_dst, mesh_src, mesh_dst, m2g_src, m2g_dst, params):
    P = params

    EP_G2M = 204800
    EP_MESH = 163840
    EP_M2G = 155648

    g2m_efeat, g2m_src, g2m_dst = _pad_edges(
        g2m_efeat, g2m_src, g2m_dst, N_MESH_, EP_G2M)
    mesh_efeat, mesh_src, mesh_dst = _pad_edges(
        mesh_efeat, mesh_src, mesh_dst, N_MESH_, EP_MESH)
    m2g_efeat, m2g_src, m2g_dst = _pad_edges(
        m2g_efeat, m2g_src, m2g_dst, N_GRID_, EP_M2G)

    W1e_g2m, W1s_g2m, W1d_g2m = _w1_split(P["g2m_edge_mlp"])
    W1e_m2g, W1s_m2g, W1d_m2g = _w1_split(P["m2g_edge_mlp"])
    proc_e = [_w1_split(P["proc_edge_%d" % i]) for i in range(L)]

    g, S_g2m = _mlp(P["grid_enc"], grid_nfeat, projs=(W1s_g2m,))
    m, D_g2m = _mlp(P["mesh_enc"], mesh_nfeat, projs=(W1d_g2m,))

    def halves(a):
        n = a.shape[0] // 2
        return a[:n], a[n:]

    def edge_stage(e_halves, efeat_halves, enc_p, S, D, src, dst, W1e,
                   edge_p, n_nodes, n_chunks):
        srcs, dsts = halves(src), halves(dst)
        parts = []
        new_e = []
        ep = dict(edge_p, W1=W1e)
        for h in range(2):
            if e_halves is None:
                e_h = _mlp(enc_p, efeat_halves[h])
            else:
                e_h = e_halves[h]
            G = _sc_gather_combine(S, D, srcs[h], dsts[h])
            e_h = _mlp(ep, e_h, adds=(G,), res=e_h)
            tb = 64 if n_chunks > 1 else TB
            parts += _sc_scatter_add(e_h, dsts[h], n_nodes, n_chunks, tb)
            new_e.append(e_h)
        return new_e, parts

    _, aggs = edge_stage(None, halves(g2m_efeat), P["g2m_edge_enc"],
                         S_g2m, D_g2m, g2m_src, g2m_dst, W1e_g2m,
                         P["g2m_edge_mlp"], N_MESH_, 1)
    nodep = P["g2m_node_mlp"]
    W1m, W1a = nodep["W1"][:H], nodep["W1"][H:]
    m, S0, D0 = _fused_mlp(
        [([m], W1m), (aggs, W1a)], [], nodep["b1"], nodep["W2"], nodep["b2"],
        nodep["g"], nodep["b"], m, [proc_e[0][1], proc_e[0][2]], 1024)

    g, D_m2g = _mlp(P["enc_grid_mlp"], g, res=g, projs=(W1d_m2g,))

    S, D = S0, D0
    e_mesh = None
    mesh_efeat_h = halves(mesh_efeat)
    for i in range(L):
        e_mesh, aggs = edge_stage(e_mesh, mesh_efeat_h,
                                  P["mesh_edge_enc"], S, D,
                                  mesh_src, mesh_dst, proc_e[i][0],
                                  P["proc_edge_%d" % i], N_MESH_, 1)
        np_ = P["proc_node_%d" % i]
        W1m, W1a = np_["W1"][:H], np_["W1"][H:]
        if i + 1 < L:
            projs = [proc_e[i + 1][1], proc_e[i + 1][2]]
        else:
            projs = [W1s_m2g]
        outs = _fused_mlp(
            [([m], W1m), (aggs, W1a)], [], np_["b1"], np_["W2"], np_["b2"],
            np_["g"], np_["b"], m, projs, 1024)
        if i + 1 < L:
            m, S, D = outs
        else:
            m, S_m2g = outs

    _, aggs = edge_stage(None, halves(m2g_efeat), P["m2g_edge_enc"],
                         S_m2g, D_m2g, m2g_src, m2g_dst, W1e_m2g,
                         P["m2g_edge_mlp"], N_GRID_, 4)
    decp = P["dec_node_mlp"]
    W1g, W1a = decp["W1"][:H], decp["W1"][H:]
    g = _fused_mlp(
        [([g], W1g), (aggs, W1a)], [], decp["b1"], decp["W2"], decp["b2"],
        decp["g"], decp["b"], g, [], 1024)

    return _mlp(P["final_mlp"], g, ln=False)


# speedup vs baseline: 3.0363x; 1.0373x over previous
import functools

import jax
import jax.numpy as jnp
from jax import lax
from jax.experimental import pallas as pl
from jax.experimental.pallas import tpu as pltpu
from jax.experimental.pallas import tpu_sc as plsc

H = 128
PREC = lax.Precision.DEFAULT
NC = 2
NS = 16
NW = NC * NS
TB = 128


def _fused_mlp(groups, adds, b1, W2, b2, lng, lnb, res, projs, block_rows):
    n_groups = len(groups)
    xs_counts = [len(xs) for xs, _ in groups]
    n_adds = len(adds)
    has_res = res is not None
    ln = lng is not None
    n_projs = len(projs)

    some_x = groups[0][0][0] if groups else adds[0]
    N = some_x.shape[0]
    dout = W2.shape[1]

    def body(*refs):
        it = iter(refs)
        u = None
        for gi in range(n_groups):
            xs = [next(it)[...] for _ in range(xs_counts[gi])]
            W = next(it)[...]
            x = xs[0]
            for extra in xs[1:]:
                x = x + extra
            t = jnp.dot(x, W, preferred_element_type=jnp.float32,
                        precision=PREC)
            u = t if u is None else u + t
        for _ in range(n_adds):
            a = next(it)[...]
            u = a if u is None else u + a
        b1v = next(it)[...]
        W2v = next(it)[...]
        b2v = next(it)[...]
        u = u + b1v
        h = u * jax.nn.sigmoid(u)
        z = jnp.dot(h, W2v, preferred_element_type=jnp.float32,
                    precision=PREC) + b2v
        if ln:
            gv = next(it)[...]
            bv = next(it)[...]
            mu = jnp.mean(z, axis=-1, keepdims=True)
            zc = z - mu
            var = jnp.mean(zc * zc, axis=-1, keepdims=True)
            z = zc * lax.rsqrt(var + 1e-5) * gv + bv
        if has_res:
            z = next(it)[...] + z
        pws = [next(it)[...] for _ in range(n_projs)]
        outs = list(it)
        outs[0][...] = z
        for k in range(n_projs):
            outs[1 + k][...] = jnp.dot(z, pws[k],
                                       preferred_element_type=jnp.float32,
                                       precision=PREC)

    inputs = []
    in_specs = []

    def add_rowblocked(a):
        inputs.append(a)
        in_specs.append(pl.BlockSpec((block_rows, a.shape[1]),
                                     lambda i: (i, 0)))

    def add_full(a):
        inputs.append(a)
        in_specs.append(pl.BlockSpec(a.shape, lambda i: (0,) * a.ndim))

    for xs, W in groups:
        for x in xs:
            add_rowblocked(x)
        add_full(W)
    for a in adds:
        add_rowblocked(a)
    add_full(b1.reshape(1, -1))
    add_full(W2)
    add_full(b2.reshape(1, -1))
    if ln:
        add_full(lng.reshape(1, -1))
        add_full(lnb.reshape(1, -1))
    if has_res:
        add_rowblocked(res)
    for Pw in projs:
        add_full(Pw)

    out_shapes = [jax.ShapeDtypeStruct((N, dout), jnp.float32)]
    out_shapes += [jax.ShapeDtypeStruct((N, H), jnp.float32)
                   for _ in range(n_projs)]
    out_specs = [pl.BlockSpec((block_rows, dout), lambda i: (i, 0))]
    out_specs += [pl.BlockSpec((block_rows, H), lambda i: (i, 0))
                  for _ in range(n_projs)]

    outs = pl.pallas_call(
        body,
        grid=(pl.cdiv(N, block_rows),),
        in_specs=in_specs,
        out_specs=out_specs,
        out_shape=out_shapes,
    )(*inputs)
    return outs if n_projs else outs[0]


def _mlp(p, x, ln=True, res=None, adds=(), extra_groups=(), projs=(),
         block_rows=1024):
    groups = [([x], p["W1"])] + list(extra_groups)
    lng = p["g"] if ln else None
    lnb = p["b"] if ln else None
    return _fused_mlp(groups, list(adds), p["b1"], p["W2"], p["b2"],
                      lng, lnb, res, list(projs), block_rows)


def _sc_gather_combine(S, D, src, dst, small):
    E = src.shape[0]
    per_w = E // NW
    nt = per_w // TB
    n_small = (S if small == "s" else D).shape[0]
    stripe = -(-(-(-n_small // NS)) // 8) * 8
    spr = -(-(n_small + 1) // 16) * 16
    mesh = plsc.VectorSubcoreMesh(core_axis_name="c", subcore_axis_name="s")

    @functools.partial(
        pl.kernel, mesh=mesh,
        out_type=jax.ShapeDtypeStruct((E, H), jnp.float32),
        scratch_types=[
            pltpu.VMEM((per_w,), jnp.int32),
            pltpu.VMEM((per_w,), jnp.int32),
            pltpu.VMEM((2, TB, H), jnp.float32),
            pltpu.VMEM_SHARED((spr, H), jnp.float32),
            pltpu.SemaphoreType.DMA,
            [pltpu.SemaphoreType.DMA] * 2,
            [pltpu.SemaphoreType.DMA] * 2,
            [pltpu.SemaphoreType.DMA] * 2,
        ],
    )
    def k(S_hbm, D_hbm, src_hbm, dst_hbm, g_hbm,
          sidx, didx, rs, tbl, semi, semg, sema, semo):
        s = lax.axis_index("s")
        wid = s * NC + lax.axis_index("c")
        base = wid * per_w

        small_hbm = S_hbm if small == "s" else D_hbm
        off = jnp.minimum(s * stripe, n_small - stripe)
        cl = pltpu.async_copy(small_hbm.at[pl.ds(off, stripe)],
                              tbl.at[pl.ds(off, stripe)], semi)
        c1 = pltpu.async_copy(src_hbm.at[pl.ds(base, per_w)], sidx, semi)
        c2 = pltpu.async_copy(dst_hbm.at[pl.ds(base, per_w)], didx, semi)
        cl.wait()
        c1.wait()
        c2.wait()
        plsc.subcore_barrier()

        s_ref = tbl if small == "s" else S_hbm
        d_ref = tbl if small == "d" else D_hbm

        def issue_gather(t, slot):
            pltpu.async_copy(s_ref.at[sidx.at[pl.ds(t * TB, TB)]],
                             rs.at[slot], semg[slot])

        def wait_gather(slot):
            pltpu.make_async_copy(s_ref.at[pl.ds(0, TB)], rs.at[slot],
                                  semg[slot]).wait()

        def issue_store(t, slot):
            pltpu.async_copy(d_ref.at[didx.at[pl.ds(t * TB, TB)]],
                             rs.at[slot], sema[slot], add=True)
            pltpu.make_async_copy(d_ref.at[pl.ds(0, TB)], rs.at[slot],
                                  sema[slot]).wait()
            pltpu.async_copy(rs.at[slot], g_hbm.at[pl.ds(base + t * TB, TB)],
                             semo[slot])

        def wait_store(slot):
            pltpu.make_async_copy(rs.at[slot], g_hbm.at[pl.ds(0, TB)],
                                  semo[slot]).wait()

        issue_gather(0, 0)

        def pair(i, carry):
            t0 = 2 * i

            @pl.when(i > 0)
            def _():
                wait_store(1)

            issue_gather(t0 + 1, 1)
            wait_gather(0)
            issue_store(t0, 0)

            @pl.when(t0 + 2 < nt)
            def _():
                wait_store(0)
                issue_gather(t0 + 2, 0)

            wait_gather(1)
            issue_store(t0 + 1, 1)
            return carry

        lax.fori_loop(0, nt // 2, pair, 0)
        if nt % 2:
            wait_gather(0)
            issue_store(nt - 1, 0)
            wait_store(0)
            if nt > 1:
                wait_store(1)
        else:
            wait_store(0)
            wait_store(1)

    return k(S, D, src, dst)


def _sc_scatter_add(e, dst, n_nodes, n_chunks, tb=TB):
    E = e.shape[0]
    per_w = E // NW
    nt = per_w // tb
    ch = -(-max(n_nodes + 1, 128) // (n_chunks * 128)) * 128
    stripe = ch // 16
    sp_rows = ch + 16
    mesh = plsc.VectorSubcoreMesh(core_axis_name="c", subcore_axis_name="s")
    zeros = jnp.zeros((ch, H), jnp.float32)

    @functools.partial(
        pl.kernel, mesh=mesh,
        out_type=jax.ShapeDtypeStruct((NC, n_chunks * ch, H), jnp.float32),
        scratch_types=[
            pltpu.VMEM((per_w,), jnp.int32),
            pltpu.VMEM((nt, tb), jnp.int32),
            pltpu.VMEM((2, tb, H), jnp.float32),
            pltpu.VMEM_SHARED((sp_rows, H), jnp.float32),
            pltpu.SemaphoreType.DMA,
            [pltpu.SemaphoreType.DMA] * 2,
            [pltpu.SemaphoreType.DMA] * 2,
        ],
    )
    def k(e_hbm, dst_hbm, z_hbm, out_hbm, didx, lidx, er, acc,
          semi, seme, sema):
        c = lax.axis_index("c")
        s = lax.axis_index("s")
        wid = s * NC + c
        base = wid * per_w

        pltpu.async_copy(dst_hbm.at[pl.ds(base, per_w)], didx, semi).wait()

        def issue_load(t, slot):
            pltpu.async_copy(e_hbm.at[pl.ds(base + t * tb, tb)],
                             er.at[slot], seme[slot])

        def wait_load(slot):
            pltpu.make_async_copy(e_hbm.at[pl.ds(0, tb)], er.at[slot],
                                  seme[slot]).wait()

        def issue_scat(t, slot):
            if n_chunks == 1:
                rows = didx.at[pl.ds(t * tb, tb)]
            else:
                rows = lidx.at[t]
            pltpu.async_copy(er.at[slot], acc.at[rows], sema[slot],
                             add=True)

        def wait_scat(slot):
            pltpu.make_async_copy(er.at[slot], acc.at[pl.ds(0, tb)],
                                  sema[slot]).wait()

        for chunk in range(n_chunks):
            cbase = chunk * ch
            pltpu.sync_copy(z_hbm.at[pl.ds(s * stripe, stripe)],
                            acc.at[pl.ds(s * stripe, stripe)])

            if n_chunks > 1:
                def trans(t, carry):
                    for j in range(tb // 16):
                        v = didx[pl.ds(t * tb + j * 16, 16)]
                        inb = (v >= cbase) & (v < cbase + ch)
                        lidx[t, pl.ds(j * 16, 16)] = jnp.where(
                            inb, v - cbase, ch)
                    return carry

                lax.fori_loop(0, nt, trans, 0)
            plsc.subcore_barrier()

            issue_load(0, 0)

            def pair(i, carry):
                t0 = 2 * i

                @pl.when(i > 0)
                def _():
                    wait_scat(1)

                issue_load(t0 + 1, 1)
                wait_load(0)
                issue_scat(t0, 0)

                @pl.when(t0 + 2 < nt)
                def _():
                    wait_scat(0)
                    issue_load(t0 + 2, 0)

                wait_load(1)
                issue_scat(t0 + 1, 1)
                return carry

            lax.fori_loop(0, nt // 2, pair, 0)
            if nt % 2:
                wait_load(0)
                issue_scat(nt - 1, 0)
                wait_scat(0)
                if nt > 1:
                    wait_scat(1)
            else:
                wait_scat(0)
                wait_scat(1)
            plsc.subcore_barrier()
            pltpu.sync_copy(
                acc.at[pl.ds(s * stripe, stripe)],
                out_hbm.at[c, pl.ds(cbase + s * stripe, stripe)])
            plsc.subcore_barrier()

    out = k(e, dst, zeros)
    return [out[0, :n_nodes], out[1, :n_nodes]]


def _pad_edges(efeat, src, dst, n_dst, e_pad):
    e = efeat.shape[0]
    pad = e_pad - e
    efeat = jnp.pad(efeat, ((0, pad), (0, 0)))
    src = jnp.pad(src, (0, pad))
    dst = jnp.pad(dst, (0, pad), constant_values=n_dst)
    return efeat, src, dst


def _w1_split(p):
    W1 = p["W1"]
    return W1[:H], W1[H:2 * H], W1[2 * H:]


L = 4
N_MESH_ = 10000
N_GRID_ = 50000


def kernel(grid_nfeat, mesh_nfeat, g2m_efeat, mesh_efeat, m2g_efeat,
           g2m_src, g2m_dst, mesh_src, mesh_dst, m2g_src, m2g_dst, params):
    P = params

    EP_G2M = 204800
    EP_MESH = 163840
    EP_M2G = 155648

    g2m_efeat, g2m_src, g2m_dst = _pad_edges(
        g2m_efeat, g2m_src, g2m_dst, N_MESH_, EP_G2M)
    mesh_efeat, mesh_src, mesh_dst = _pad_edges(
        mesh_efeat, mesh_src, mesh_dst, N_MESH_, EP_MESH)
    m2g_efeat, m2g_src, m2g_dst = _pad_edges(
        m2g_efeat, m2g_src, m2g_dst, N_GRID_, EP_M2G)

    W1e_g2m, W1s_g2m, W1d_g2m = _w1_split(P["g2m_edge_mlp"])
    W1e_m2g, W1s_m2g, W1d_m2g = _w1_split(P["m2g_edge_mlp"])
    proc_e = [_w1_split(P["proc_edge_%d" % i]) for i in range(L)]

    g, S_g2m = _mlp(P["grid_enc"], grid_nfeat, projs=(W1s_g2m,))
    m, D_g2m = _mlp(P["mesh_enc"], mesh_nfeat, projs=(W1d_g2m,))

    def halves(a):
        n = a.shape[0] // 2
        return a[:n], a[n:]

    def edge_stage(e_halves, efeat_halves, enc_p, S, D, src, dst, W1e,
                   edge_p, n_nodes, n_chunks, small):
        srcs, dsts = halves(src), halves(dst)
        parts = []
        new_e = []
        ep = dict(edge_p, W1=W1e)
        for h in range(2):
            if e_halves is None:
                e_h = _mlp(enc_p, efeat_halves[h])
            else:
                e_h = e_halves[h]
            G = _sc_gather_combine(S, D, srcs[h], dsts[h], small)
            e_h = _mlp(ep, e_h, adds=(G,), res=e_h)
            tb = 64 if n_chunks > 1 else TB
            parts += _sc_scatter_add(e_h, dsts[h], n_nodes, n_chunks, tb)
            new_e.append(e_h)
        return new_e, parts

    _, aggs = edge_stage(None, halves(g2m_efeat), P["g2m_edge_enc"],
                         S_g2m, D_g2m, g2m_src, g2m_dst, W1e_g2m,
                         P["g2m_edge_mlp"], N_MESH_, 1, "d")
    nodep = P["g2m_node_mlp"]
    W1m, W1a = nodep["W1"][:H], nodep["W1"][H:]
    m, S0, D0 = _fused_mlp(
        [([m], W1m), (aggs, W1a)], [], nodep["b1"], nodep["W2"], nodep["b2"],
        nodep["g"], nodep["b"], m, [proc_e[0][1], proc_e[0][2]], 1024)

    g, D_m2g = _mlp(P["enc_grid_mlp"], g, res=g, projs=(W1d_m2g,))

    S, D = S0, D0
    e_mesh = None
    mesh_efeat_h = halves(mesh_efeat)
    for i in range(L):
        e_mesh, aggs = edge_stage(e_mesh, mesh_efeat_h,
                                  P["mesh_edge_enc"], S, D,
                                  mesh_src, mesh_dst, proc_e[i][0],
                                  P["proc_edge_%d" % i], N_MESH_, 1,
                                  "s")
        np_ = P["proc_node_%d" % i]
        W1m, W1a = np_["W1"][:H], np_["W1"][H:]
        if i + 1 < L:
            projs = [proc_e[i + 1][1], proc_e[i + 1][2]]
        else:
            projs = [W1s_m2g]
        outs = _fused_mlp(
            [([m], W1m), (aggs, W1a)], [], np_["b1"], np_["W2"], np_["b2"],
            np_["g"], np_["b"], m, projs, 1024)
        if i + 1 < L:
            m, S, D = outs
        else:
            m, S_m2g = outs

    _, aggs = edge_stage(None, halves(m2g_efeat), P["m2g_edge_enc"],
                         S_m2g, D_m2g, m2g_src, m2g_dst, W1e_m2g,
                         P["m2g_edge_mlp"], N_GRID_, 4, "s")
    decp = P["dec_node_mlp"]
    W1g, W1a = decp["W1"][:H], decp["W1"][H:]
    g = _fused_mlp(
        [([g], W1g), (aggs, W1a)], [], decp["b1"], decp["W2"], decp["b2"],
        decp["g"], decp["b"], g, [], 1024)

    return _mlp(P["final_mlp"], g, ln=False)
